# baseline (speedup 1.0000x reference)
import functools
import numpy as np
import jax
import jax.numpy as jnp
from jax import lax
from jax.experimental import pallas as pl
from jax.experimental.pallas import tpu as pltpu
from jax.experimental.pallas import tpu_sc as plsc

N_NODES = 10000
E_EDGES = 160000
OUT, HEADS, CPH = 256, 8, 32
D_EDGE = 16
ACC_W = 160

_NC, _NS = 2, 16
_NW = _NC * _NS
_GC = 200
_EW = E_EDGES // _NW
_SCC = 400
_ET = E_EDGES // _NS
_STR = 640
_STR_LAST = N_NODES - 15 * _STR

_BLK = np.zeros((OUT, HEADS), np.float32)
for _h in range(HEADS):
    _BLK[_h * CPH:(_h + 1) * CPH, _h] = 1.0
_BLKT = _BLK.T.copy()
_P0 = np.zeros((HEADS, 32), np.float32)
_P1 = np.zeros((HEADS, 32), np.float32)
for _h in range(4):
    _P0[_h, _h] = 1.0
    _P1[_h + 4, _h] = 1.0
_B32 = np.zeros((32, 128), np.float32)
for _h in range(4):
    _B32[_h, _h * CPH:(_h + 1) * CPH] = 1.0


def _pack_bf16_pair(y):
    lo = lax.bitcast_convert_type(y[:, :128].astype(jnp.bfloat16), jnp.uint16)
    hi = lax.bitcast_convert_type(y[:, 128:].astype(jnp.bfloat16), jnp.uint16)
    word = lo.astype(jnp.uint32) | (hi.astype(jnp.uint32) << 16)
    return lax.bitcast_convert_type(word, jnp.float32)


def _unpack_bf16_pair(p):
    u = lax.bitcast_convert_type(p, jnp.uint32)
    lo = lax.bitcast_convert_type((u & 0xFFFF).astype(jnp.uint16), jnp.bfloat16)
    hi = lax.bitcast_convert_type((u >> 16).astype(jnp.uint16), jnp.bfloat16)
    return lo.astype(jnp.float32), hi.astype(jnp.float32)


def _mm2_body(x_ref, wl_ref, wr_ref, ol_ref, or_ref):
    x = x_ref[...]
    ol_ref[...] = _pack_bf16_pair(jnp.dot(x, wl_ref[...], preferred_element_type=jnp.float32))
    or_ref[...] = _pack_bf16_pair(jnp.dot(x, wr_ref[...], preferred_element_type=jnp.float32))


def _mm2(x, Wl, Wr):
    M, K = x.shape
    BM = 400
    return pl.pallas_call(
        _mm2_body,
        grid=(M // BM,),
        in_specs=[
            pl.BlockSpec((BM, K), lambda i: (i, 0)),
            pl.BlockSpec((K, OUT), lambda i: (0, 0)),
            pl.BlockSpec((K, OUT), lambda i: (0, 0)),
        ],
        out_specs=[
            pl.BlockSpec((BM, 128), lambda i: (i, 0)),
            pl.BlockSpec((BM, 128), lambda i: (i, 0)),
        ],
        out_shape=[
            jax.ShapeDtypeStruct((M, 128), jnp.float32),
            jax.ShapeDtypeStruct((M, 128), jnp.float32),
        ],
    )(x, Wl, Wr)


def _edge_body(xs_ref, xd_ref, ea_ref, we_ref, a_ref, bt_ref, p0_ref, p1_ref,
               r0_ref, r1_ref):
    xs0, xs1 = _unpack_bf16_pair(xs_ref[...])
    xd0, xd1 = _unpack_bf16_pair(xd_ref[...])
    e = jnp.dot(ea_ref[...], we_ref[...], preferred_element_type=jnp.float32)
    m0 = xs0 + xd0 + e[:, :128]
    m1 = xs1 + xd1 + e[:, 128:]
    m0 = jnp.where(m0 >= 0, m0, 0.2 * m0)
    m1 = jnp.where(m1 >= 0, m1, 0.2 * m1)
    alpha = (jnp.dot(m0, a_ref[:128, :], preferred_element_type=jnp.float32)
             + jnp.dot(m1, a_ref[128:, :], preferred_element_type=jnp.float32))
    w = jnp.exp(alpha)
    wb0 = jnp.dot(w, bt_ref[:, :128], preferred_element_type=jnp.float32)
    wb1 = jnp.dot(w, bt_ref[:, 128:], preferred_element_type=jnp.float32)
    w0 = jnp.dot(w, p0_ref[...], preferred_element_type=jnp.float32)
    w1 = jnp.dot(w, p1_ref[...], preferred_element_type=jnp.float32)
    r0_ref[...] = jnp.concatenate([xs0 * wb0, w0], axis=1).astype(jnp.bfloat16)
    r1_ref[...] = jnp.concatenate([xs1 * wb1, w1], axis=1).astype(jnp.bfloat16)


def _edge(XS, XD, ea, We, att):
    BE = 1600
    A = att.reshape(-1)[:, None] * jnp.asarray(_BLK)
    return pl.pallas_call(
        _edge_body,
        grid=(E_EDGES // BE,),
        in_specs=[
            pl.BlockSpec((BE, 128), lambda i: (i, 0)),
            pl.BlockSpec((BE, 128), lambda i: (i, 0)),
            pl.BlockSpec((BE, D_EDGE), lambda i: (i, 0)),
            pl.BlockSpec((D_EDGE, OUT), lambda i: (0, 0)),
            pl.BlockSpec((OUT, HEADS), lambda i: (0, 0)),
            pl.BlockSpec((HEADS, OUT), lambda i: (0, 0)),
            pl.BlockSpec((HEADS, 32), lambda i: (0, 0)),
            pl.BlockSpec((HEADS, 32), lambda i: (0, 0)),
        ],
        out_specs=[
            pl.BlockSpec((BE, ACC_W), lambda i: (i, 0)),
            pl.BlockSpec((BE, ACC_W), lambda i: (i, 0)),
        ],
        out_shape=[
            jax.ShapeDtypeStruct((E_EDGES, ACC_W), jnp.bfloat16),
            jax.ShapeDtypeStruct((E_EDGES, ACC_W), jnp.bfloat16),
        ],
    )(XS, XD, ea, We, A, jnp.asarray(_BLKT), jnp.asarray(_P0), jnp.asarray(_P1))


def _finish_body(a0_ref, a1_ref, b16_ref, b_ref, lw_ref, lb_ref, o_ref):
    db0 = jnp.dot(a0_ref[:, 128:160].astype(jnp.float32), b16_ref[...],
                  preferred_element_type=jnp.float32)
    db1 = jnp.dot(a1_ref[:, 128:160].astype(jnp.float32), b16_ref[...],
                  preferred_element_type=jnp.float32)
    x0 = a0_ref[:, 0:128].astype(jnp.float32) / (db0 + 1e-16)
    x1 = a1_ref[:, 0:128].astype(jnp.float32) / (db1 + 1e-16)
    x = jnp.concatenate([x0, x1], axis=1) + b_ref[...]
    y = jnp.dot(x, lw_ref[...], preferred_element_type=jnp.float32) + lb_ref[...]
    o_ref[...] = x + jnp.where(y > 0, y, jnp.exp(jnp.minimum(y, 0.0)) - 1.0)


def _finish(acc0, acc1, bias, lW, lb):
    BM = 400
    return pl.pallas_call(
        _finish_body,
        grid=(N_NODES // BM,),
        in_specs=[
            pl.BlockSpec((BM, ACC_W), lambda i: (i, 0)),
            pl.BlockSpec((BM, ACC_W), lambda i: (i, 0)),
            pl.BlockSpec((32, 128), lambda i: (0, 0)),
            pl.BlockSpec((1, OUT), lambda i: (0, 0)),
            pl.BlockSpec((OUT, OUT), lambda i: (0, 0)),
            pl.BlockSpec((1, OUT), lambda i: (0, 0)),
        ],
        out_specs=pl.BlockSpec((BM, OUT), lambda i: (i, 0)),
        out_shape=jax.ShapeDtypeStruct((N_NODES, OUT), jnp.float32),
    )(acc0, acc1, jnp.asarray(_B32), bias.reshape(1, OUT), lW, lb.reshape(1, OUT))


def _head_body(xm_ref, xp_ref, f1m_ref, f1p_ref, f1b_ref, f2w_ref, f2b_ref,
               o_ref, sm_ref, sp_ref):
    i = pl.program_id(0)

    @pl.when(i == 0)
    def _():
        sm_ref[...] = jnp.zeros_like(sm_ref)
        sp_ref[...] = jnp.zeros_like(sp_ref)

    sm_ref[...] += jnp.sum(xm_ref[...], 0, keepdims=True)
    sp_ref[...] += jnp.sum(xp_ref[...], 0, keepdims=True)

    @pl.when(i == pl.num_programs(0) - 1)
    def _():
        z = (jnp.dot(sm_ref[...], f1m_ref[...], preferred_element_type=jnp.float32)
             + jnp.dot(sp_ref[...], f1p_ref[...], preferred_element_type=jnp.float32)
             + f1b_ref[...])
        z = jnp.where(z > 0, z, jnp.exp(jnp.minimum(z, 0.0)) - 1.0)
        t = jnp.dot(z, f2w_ref[...], preferred_element_type=jnp.float32) + f2b_ref[...]
        o_ref[...] = 1.0 / (1.0 + jnp.exp(-t[0:1, 0:1]))


def _head(x_m, x_p, f1W, f1b, f2W, f2b):
    BM = 400
    return pl.pallas_call(
        _head_body,
        grid=(N_NODES // BM,),
        in_specs=[
            pl.BlockSpec((BM, OUT), lambda i: (i, 0)),
            pl.BlockSpec((BM, OUT), lambda i: (i, 0)),
            pl.BlockSpec((OUT, OUT), lambda i: (0, 0)),
            pl.BlockSpec((OUT, OUT), lambda i: (0, 0)),
            pl.BlockSpec((1, OUT), lambda i: (0, 0)),
            pl.BlockSpec((OUT, 1), lambda i: (0, 0)),
            pl.BlockSpec((1, 1), lambda i: (0, 0)),
        ],
        out_specs=pl.BlockSpec((1, 1), lambda i: (0, 0)),
        out_shape=jax.ShapeDtypeStruct((1, 1), jnp.float32),
        scratch_shapes=[
            pltpu.VMEM((1, OUT), jnp.float32),
            pltpu.VMEM((1, OUT), jnp.float32),
        ],
    )(x_m, x_p, f1W[:OUT], f1W[OUT:], f1b.reshape(1, OUT), f2W, f2b.reshape(1, 1))


def _sc_gather2(xl, xr, src, dst):
    mesh = plsc.VectorSubcoreMesh(core_axis_name="c", subcore_axis_name="s")

    @functools.partial(
        pl.kernel,
        out_type=[jax.ShapeDtypeStruct((E_EDGES, 128), jnp.float32),
                  jax.ShapeDtypeStruct((E_EDGES, 128), jnp.float32)],
        mesh=mesh,
        scratch_types=[
            pltpu.VMEM((_GC,), jnp.int32),
            pltpu.VMEM((_GC,), jnp.int32),
            pltpu.VMEM((_GC, 128), jnp.float32),
            pltpu.VMEM((_GC, 128), jnp.float32),
            pltpu.SemaphoreType.DMA,
            pltpu.SemaphoreType.DMA,
        ],
    )
    def k(xl_hbm, xr_hbm, src_hbm, dst_hbm, xs_hbm, xd_hbm,
          sidx, didx, srows, drows, sem1, sem2):
        wid = lax.axis_index("s") * _NC + lax.axis_index("c")
        w0 = wid * _EW

        def body(i, _):
            base = w0 + i * _GC
            pltpu.sync_copy(src_hbm.at[pl.ds(base, _GC)], sidx)
            pltpu.sync_copy(dst_hbm.at[pl.ds(base, _GC)], didx)
            a = pltpu.async_copy(xl_hbm.at[sidx], srows, sem1)
            b = pltpu.async_copy(xr_hbm.at[didx], drows, sem2)
            a.wait()
            pltpu.sync_copy(srows, xs_hbm.at[pl.ds(base, _GC)])
            b.wait()
            pltpu.sync_copy(drows, xd_hbm.at[pl.ds(base, _GC)])
            return 0

        lax.fori_loop(0, _EW // _GC, body, 0)

    return k(xl, xr, src, dst)


def _sc_scatter(R0, R1, dst, zrows):
    mesh = plsc.VectorSubcoreMesh(core_axis_name="c", subcore_axis_name="s")

    @functools.partial(
        pl.kernel,
        out_type=[jax.ShapeDtypeStruct((N_NODES, ACC_W), jnp.bfloat16),
                  jax.ShapeDtypeStruct((N_NODES, ACC_W), jnp.bfloat16)],
        mesh=mesh,
        scratch_types=[
            pltpu.VMEM((_SCC,), jnp.int32),
            pltpu.VMEM((_SCC, ACC_W), jnp.bfloat16),
            pltpu.VMEM_SHARED((N_NODES, ACC_W), jnp.bfloat16),
            pltpu.SemaphoreType.DMA,
        ],
        compiler_params=pltpu.CompilerParams(use_tc_tiling_on_sc=False),
    )
    def k(r0_hbm, r1_hbm, dst_hbm, z_hbm, a0_hbm, a1_hbm, idxb, valb, acc, sem):
        sc = lax.axis_index("c")
        t = lax.axis_index("s")

        @pl.when(t < 15)
        def _():
            pltpu.sync_copy(z_hbm, acc.at[pl.ds(t * _STR, _STR)])

        @pl.when(t == 15)
        def _():
            pltpu.sync_copy(z_hbm.at[pl.ds(0, _STR_LAST)],
                            acc.at[pl.ds(15 * _STR, _STR_LAST)])

        plsc.subcore_barrier()

        def body(i, _):
            base = t * _ET + i * _SCC
            pltpu.sync_copy(dst_hbm.at[pl.ds(base, _SCC)], idxb)

            @pl.when(sc == 0)
            def _():
                pltpu.sync_copy(r0_hbm.at[pl.ds(base, _SCC)], valb)

            @pl.when(sc == 1)
            def _():
                pltpu.sync_copy(r1_hbm.at[pl.ds(base, _SCC)], valb)

            pltpu.sync_copy(valb, acc.at[idxb], add=True)
            return 0

        lax.fori_loop(0, _ET // _SCC, body, 0)
        plsc.subcore_barrier()

        def wb(out_hbm):
            @pl.when(t < 15)
            def _():
                pltpu.sync_copy(acc.at[pl.ds(t * _STR, _STR)],
                                out_hbm.at[pl.ds(t * _STR, _STR)])

            @pl.when(t == 15)
            def _():
                pltpu.sync_copy(acc.at[pl.ds(15 * _STR, _STR_LAST)],
                                out_hbm.at[pl.ds(15 * _STR, _STR_LAST)])

        @pl.when(sc == 0)
        def _():
            wb(a0_hbm)

        @pl.when(sc == 1)
        def _():
            wb(a1_hbm)

    return k(R0, R1, dst, zrows)


def _gat_layer(x, src, dst, ea, zrows, Wl, Wr, We, att, bias, lW, lb):
    xl, xr = _mm2(x, Wl, Wr)
    XS, XD = _sc_gather2(xl, xr, src, dst)
    R0, R1 = _edge(XS, XD, ea, We, att)
    acc0, acc1 = _sc_scatter(R0, R1, dst, zrows)
    return _finish(acc0, acc1, bias, lW, lb)


def _two_blocks(x_m, mg_ei, mg_ea, mg_layers, x_p, pg_ei, pg_ea, pg_layers, zrows):
    m_src, m_dst = mg_ei[0], mg_ei[1]
    p_src, p_dst = pg_ei[0], pg_ei[1]
    for lm, lp in zip(mg_layers, pg_layers):
        (mWl, mWr, mWe, matt, mb, mlW, mlb) = lm
        (pWl, pWr, pWe, patt, pb, plW, plb) = lp
        m_xl, m_xr = _mm2(x_m, mWl, mWr)
        p_xl, p_xr = _mm2(x_p, pWl, pWr)
        m_XS, m_XD = _sc_gather2(m_xl, m_xr, m_src, m_dst)
        p_XS, p_XD = _sc_gather2(p_xl, p_xr, p_src, p_dst)
        m_R0, m_R1 = _edge(m_XS, m_XD, mg_ea, mWe, matt)
        p_R0, p_R1 = _edge(p_XS, p_XD, pg_ea, pWe, patt)
        m_a0, m_a1 = _sc_scatter(m_R0, m_R1, m_dst, zrows)
        p_a0, p_a1 = _sc_scatter(p_R0, p_R1, p_dst, zrows)
        x_m = _finish(m_a0, m_a1, mb, mlW, mlb)
        x_p = _finish(p_a0, p_a1, pb, plW, plb)
    return x_m, x_p


def kernel(mg_x, mg_edge_index, mg_edge_attr, pg_x, pg_edge_index, pg_edge_attr,
           mg_Wl0, mg_Wr0, mg_We0, mg_att0, mg_b0, mg_lW0, mg_lb0,
           mg_Wl1, mg_Wr1, mg_We1, mg_att1, mg_b1, mg_lW1, mg_lb1,
           pg_Wl0, pg_Wr0, pg_We0, pg_att0, pg_b0, pg_lW0, pg_lb0,
           pg_Wl1, pg_Wr1, pg_We1, pg_att1, pg_b1, pg_lW1, pg_lb1,
           f1W, f1b, f2W, f2b):
    mg_layers = [
        (mg_Wl0, mg_Wr0, mg_We0, mg_att0, mg_b0, mg_lW0, mg_lb0),
        (mg_Wl1, mg_Wr1, mg_We1, mg_att1, mg_b1, mg_lW1, mg_lb1),
    ]
    pg_layers = [
        (pg_Wl0, pg_Wr0, pg_We0, pg_att0, pg_b0, pg_lW0, pg_lb0),
        (pg_Wl1, pg_Wr1, pg_We1, pg_att1, pg_b1, pg_lW1, pg_lb1),
    ]
    zrows = jnp.zeros((_STR, ACC_W), jnp.bfloat16)
    x_m, x_p = _two_blocks(mg_x, mg_edge_index, mg_edge_attr, mg_layers,
                           pg_x, pg_edge_index, pg_edge_attr, pg_layers, zrows)
    return _head(x_m, x_p, f1W, f1b, f2W, f2b)


# speedup vs baseline: 1.0816x; 1.0816x over previous
import functools
import numpy as np
import jax
import jax.numpy as jnp
from jax import lax
from jax.experimental import pallas as pl
from jax.experimental.pallas import tpu as pltpu
from jax.experimental.pallas import tpu_sc as plsc

N_NODES = 10000
E_EDGES = 160000
OUT, HEADS, CPH = 256, 8, 32
D_EDGE = 16
ACC_W = 144

_NC, _NS = 2, 16
_NW = _NC * _NS
_GC = 200
_EW = E_EDGES // _NW
_SCC = 200
_ET = E_EDGES // _NS
_STR = 640
_STR_LAST = N_NODES - 15 * _STR

_BLK = np.zeros((OUT, HEADS), np.float32)
for _h in range(HEADS):
    _BLK[_h * CPH:(_h + 1) * CPH, _h] = 1.0
_BLKT = _BLK.T.copy()
_P0 = np.zeros((HEADS, 16), np.float32)
_P1 = np.zeros((HEADS, 16), np.float32)
for _h in range(4):
    _P0[_h, _h] = 1.0
    _P1[_h + 4, _h] = 1.0
_B16 = np.zeros((16, 128), np.float32)
for _h in range(4):
    _B16[_h, _h * CPH:(_h + 1) * CPH] = 1.0


def _pack_bf16_pair(y):
    lo = lax.bitcast_convert_type(y[:, :128].astype(jnp.bfloat16), jnp.uint16)
    hi = lax.bitcast_convert_type(y[:, 128:].astype(jnp.bfloat16), jnp.uint16)
    word = lo.astype(jnp.uint32) | (hi.astype(jnp.uint32) << 16)
    return lax.bitcast_convert_type(word, jnp.float32)


def _unpack_bf16_pair(p):
    u = lax.bitcast_convert_type(p, jnp.uint32)
    lo = lax.bitcast_convert_type((u & 0xFFFF).astype(jnp.uint16), jnp.bfloat16)
    hi = lax.bitcast_convert_type((u >> 16).astype(jnp.uint16), jnp.bfloat16)
    return lo.astype(jnp.float32), hi.astype(jnp.float32)


def _mm2_body(x_ref, wl_ref, wr_ref, ol_ref, or_ref):
    x = x_ref[...]
    ol_ref[...] = _pack_bf16_pair(jnp.dot(x, wl_ref[...], preferred_element_type=jnp.float32))
    or_ref[...] = _pack_bf16_pair(jnp.dot(x, wr_ref[...], preferred_element_type=jnp.float32))


def _mm2(x, Wl, Wr):
    M, K = x.shape
    BM = 400
    return pl.pallas_call(
        _mm2_body,
        grid=(M // BM,),
        in_specs=[
            pl.BlockSpec((BM, K), lambda i: (i, 0)),
            pl.BlockSpec((K, OUT), lambda i: (0, 0)),
            pl.BlockSpec((K, OUT), lambda i: (0, 0)),
        ],
        out_specs=[
            pl.BlockSpec((BM, 128), lambda i: (i, 0)),
            pl.BlockSpec((BM, 128), lambda i: (i, 0)),
        ],
        out_shape=[
            jax.ShapeDtypeStruct((M, 128), jnp.float32),
            jax.ShapeDtypeStruct((M, 128), jnp.float32),
        ],
    )(x, Wl, Wr)


def _edge_body(xs_ref, xd_ref, ea_ref, we_ref, a_ref, bt_ref, p0_ref, p1_ref,
               r0_ref, r1_ref):
    xs0, xs1 = _unpack_bf16_pair(xs_ref[...])
    xd0, xd1 = _unpack_bf16_pair(xd_ref[...])
    e = jnp.dot(ea_ref[...], we_ref[...], preferred_element_type=jnp.float32)
    m0 = xs0 + xd0 + e[:, :128]
    m1 = xs1 + xd1 + e[:, 128:]
    m0 = jnp.where(m0 >= 0, m0, 0.2 * m0)
    m1 = jnp.where(m1 >= 0, m1, 0.2 * m1)
    alpha = (jnp.dot(m0, a_ref[:128, :], preferred_element_type=jnp.float32)
             + jnp.dot(m1, a_ref[128:, :], preferred_element_type=jnp.float32))
    w = jnp.exp(alpha)
    wb0 = jnp.dot(w, bt_ref[:, :128], preferred_element_type=jnp.float32)
    wb1 = jnp.dot(w, bt_ref[:, 128:], preferred_element_type=jnp.float32)
    w0 = jnp.dot(w, p0_ref[...], preferred_element_type=jnp.float32)
    w1 = jnp.dot(w, p1_ref[...], preferred_element_type=jnp.float32)
    r0_ref[...] = jnp.concatenate([xs0 * wb0, w0], axis=1)
    r1_ref[...] = jnp.concatenate([xs1 * wb1, w1], axis=1)


def _edge(XS, XD, ea, We, att):
    BE = 1600
    A = att.reshape(-1)[:, None] * jnp.asarray(_BLK)
    return pl.pallas_call(
        _edge_body,
        grid=(E_EDGES // BE,),
        in_specs=[
            pl.BlockSpec((BE, 128), lambda i: (i, 0)),
            pl.BlockSpec((BE, 128), lambda i: (i, 0)),
            pl.BlockSpec((BE, D_EDGE), lambda i: (i, 0)),
            pl.BlockSpec((D_EDGE, OUT), lambda i: (0, 0)),
            pl.BlockSpec((OUT, HEADS), lambda i: (0, 0)),
            pl.BlockSpec((HEADS, OUT), lambda i: (0, 0)),
            pl.BlockSpec((HEADS, 16), lambda i: (0, 0)),
            pl.BlockSpec((HEADS, 16), lambda i: (0, 0)),
        ],
        out_specs=[
            pl.BlockSpec((BE, ACC_W), lambda i: (i, 0)),
            pl.BlockSpec((BE, ACC_W), lambda i: (i, 0)),
        ],
        out_shape=[
            jax.ShapeDtypeStruct((E_EDGES, ACC_W), jnp.float32),
            jax.ShapeDtypeStruct((E_EDGES, ACC_W), jnp.float32),
        ],
    )(XS, XD, ea, We, A, jnp.asarray(_BLKT), jnp.asarray(_P0), jnp.asarray(_P1))


def _finish_body(a0_ref, a1_ref, b16_ref, b_ref, lw_ref, lb_ref, o_ref):
    db0 = jnp.dot(a0_ref[:, 128:144], b16_ref[...], preferred_element_type=jnp.float32)
    db1 = jnp.dot(a1_ref[:, 128:144], b16_ref[...], preferred_element_type=jnp.float32)
    x0 = a0_ref[:, 0:128] / (db0 + 1e-16)
    x1 = a1_ref[:, 0:128] / (db1 + 1e-16)
    x = jnp.concatenate([x0, x1], axis=1) + b_ref[...]
    y = jnp.dot(x, lw_ref[...], preferred_element_type=jnp.float32) + lb_ref[...]
    o_ref[...] = x + jnp.where(y > 0, y, jnp.exp(jnp.minimum(y, 0.0)) - 1.0)


def _finish(acc0, acc1, bias, lW, lb):
    BM = 400
    return pl.pallas_call(
        _finish_body,
        grid=(N_NODES // BM,),
        in_specs=[
            pl.BlockSpec((BM, ACC_W), lambda i: (i, 0)),
            pl.BlockSpec((BM, ACC_W), lambda i: (i, 0)),
            pl.BlockSpec((16, 128), lambda i: (0, 0)),
            pl.BlockSpec((1, OUT), lambda i: (0, 0)),
            pl.BlockSpec((OUT, OUT), lambda i: (0, 0)),
            pl.BlockSpec((1, OUT), lambda i: (0, 0)),
        ],
        out_specs=pl.BlockSpec((BM, OUT), lambda i: (i, 0)),
        out_shape=jax.ShapeDtypeStruct((N_NODES, OUT), jnp.float32),
    )(acc0, acc1, jnp.asarray(_B16), bias.reshape(1, OUT), lW, lb.reshape(1, OUT))


def _head_body(xm_ref, xp_ref, f1m_ref, f1p_ref, f1b_ref, f2w_ref, f2b_ref,
               o_ref, sm_ref, sp_ref):
    i = pl.program_id(0)

    @pl.when(i == 0)
    def _():
        sm_ref[...] = jnp.zeros_like(sm_ref)
        sp_ref[...] = jnp.zeros_like(sp_ref)

    sm_ref[...] += jnp.sum(xm_ref[...], 0, keepdims=True)
    sp_ref[...] += jnp.sum(xp_ref[...], 0, keepdims=True)

    @pl.when(i == pl.num_programs(0) - 1)
    def _():
        z = (jnp.dot(sm_ref[...], f1m_ref[...], preferred_element_type=jnp.float32)
             + jnp.dot(sp_ref[...], f1p_ref[...], preferred_element_type=jnp.float32)
             + f1b_ref[...])
        z = jnp.where(z > 0, z, jnp.exp(jnp.minimum(z, 0.0)) - 1.0)
        t = jnp.dot(z, f2w_ref[...], preferred_element_type=jnp.float32) + f2b_ref[...]
        o_ref[...] = 1.0 / (1.0 + jnp.exp(-t[0:1, 0:1]))


def _head(x_m, x_p, f1W, f1b, f2W, f2b):
    BM = 400
    return pl.pallas_call(
        _head_body,
        grid=(N_NODES // BM,),
        in_specs=[
            pl.BlockSpec((BM, OUT), lambda i: (i, 0)),
            pl.BlockSpec((BM, OUT), lambda i: (i, 0)),
            pl.BlockSpec((OUT, OUT), lambda i: (0, 0)),
            pl.BlockSpec((OUT, OUT), lambda i: (0, 0)),
            pl.BlockSpec((1, OUT), lambda i: (0, 0)),
            pl.BlockSpec((OUT, 1), lambda i: (0, 0)),
            pl.BlockSpec((1, 1), lambda i: (0, 0)),
        ],
        out_specs=pl.BlockSpec((1, 1), lambda i: (0, 0)),
        out_shape=jax.ShapeDtypeStruct((1, 1), jnp.float32),
        scratch_shapes=[
            pltpu.VMEM((1, OUT), jnp.float32),
            pltpu.VMEM((1, OUT), jnp.float32),
        ],
    )(x_m, x_p, f1W[:OUT], f1W[OUT:], f1b.reshape(1, OUT), f2W, f2b.reshape(1, 1))


def _sc_gather2(xl, xr, src, dst):
    mesh = plsc.VectorSubcoreMesh(core_axis_name="c", subcore_axis_name="s")

    @functools.partial(
        pl.kernel,
        out_type=[jax.ShapeDtypeStruct((E_EDGES, 128), jnp.float32),
                  jax.ShapeDtypeStruct((E_EDGES, 128), jnp.float32)],
        mesh=mesh,
        scratch_types=[
            pltpu.VMEM((_GC,), jnp.int32),
            pltpu.VMEM((_GC,), jnp.int32),
            pltpu.VMEM((_GC, 128), jnp.float32),
            pltpu.VMEM((_GC, 128), jnp.float32),
            pltpu.SemaphoreType.DMA,
            pltpu.SemaphoreType.DMA,
        ],
    )
    def k(xl_hbm, xr_hbm, src_hbm, dst_hbm, xs_hbm, xd_hbm,
          sidx, didx, srows, drows, sem1, sem2):
        wid = lax.axis_index("s") * _NC + lax.axis_index("c")
        w0 = wid * _EW

        def body(i, _):
            base = w0 + i * _GC
            pltpu.sync_copy(src_hbm.at[pl.ds(base, _GC)], sidx)
            pltpu.sync_copy(dst_hbm.at[pl.ds(base, _GC)], didx)
            a = pltpu.async_copy(xl_hbm.at[sidx], srows, sem1)
            b = pltpu.async_copy(xr_hbm.at[didx], drows, sem2)
            a.wait()
            pltpu.sync_copy(srows, xs_hbm.at[pl.ds(base, _GC)])
            b.wait()
            pltpu.sync_copy(drows, xd_hbm.at[pl.ds(base, _GC)])
            return 0

        lax.fori_loop(0, _EW // _GC, body, 0)

    return k(xl, xr, src, dst)


def _sc_scatter(R0, R1, dst, zrows):
    mesh = plsc.VectorSubcoreMesh(core_axis_name="c", subcore_axis_name="s")

    @functools.partial(
        pl.kernel,
        out_type=[jax.ShapeDtypeStruct((N_NODES, ACC_W), jnp.float32),
                  jax.ShapeDtypeStruct((N_NODES, ACC_W), jnp.float32)],
        mesh=mesh,
        scratch_types=[
            pltpu.VMEM((_SCC,), jnp.int32),
            pltpu.VMEM((_SCC, ACC_W), jnp.float32),
            pltpu.VMEM_SHARED((N_NODES, ACC_W), jnp.float32),
            pltpu.SemaphoreType.DMA,
        ],
        compiler_params=pltpu.CompilerParams(use_tc_tiling_on_sc=False),
    )
    def k(r0_hbm, r1_hbm, dst_hbm, z_hbm, a0_hbm, a1_hbm, idxb, valb, acc, sem):
        sc = lax.axis_index("c")
        t = lax.axis_index("s")

        @pl.when(t < 15)
        def _():
            pltpu.sync_copy(z_hbm, acc.at[pl.ds(t * _STR, _STR)])

        @pl.when(t == 15)
        def _():
            pltpu.sync_copy(z_hbm.at[pl.ds(0, _STR_LAST)],
                            acc.at[pl.ds(15 * _STR, _STR_LAST)])

        plsc.subcore_barrier()

        def body(i, _):
            base = t * _ET + i * _SCC
            pltpu.sync_copy(dst_hbm.at[pl.ds(base, _SCC)], idxb)

            @pl.when(sc == 0)
            def _():
                pltpu.sync_copy(r0_hbm.at[pl.ds(base, _SCC)], valb)

            @pl.when(sc == 1)
            def _():
                pltpu.sync_copy(r1_hbm.at[pl.ds(base, _SCC)], valb)

            pltpu.sync_copy(valb, acc.at[idxb], add=True)
            return 0

        lax.fori_loop(0, _ET // _SCC, body, 0)
        plsc.subcore_barrier()

        def wb(out_hbm):
            @pl.when(t < 15)
            def _():
                pltpu.sync_copy(acc.at[pl.ds(t * _STR, _STR)],
                                out_hbm.at[pl.ds(t * _STR, _STR)])

            @pl.when(t == 15)
            def _():
                pltpu.sync_copy(acc.at[pl.ds(15 * _STR, _STR_LAST)],
                                out_hbm.at[pl.ds(15 * _STR, _STR_LAST)])

        @pl.when(sc == 0)
        def _():
            wb(a0_hbm)

        @pl.when(sc == 1)
        def _():
            wb(a1_hbm)

    return k(R0, R1, dst, zrows)


def _gat_layer(x, src, dst, ea, zrows, Wl, Wr, We, att, bias, lW, lb):
    xl, xr = _mm2(x, Wl, Wr)
    XS, XD = _sc_gather2(xl, xr, src, dst)
    R0, R1 = _edge(XS, XD, ea, We, att)
    acc0, acc1 = _sc_scatter(R0, R1, dst, zrows)
    return _finish(acc0, acc1, bias, lW, lb)


def _two_blocks(x_m, mg_ei, mg_ea, mg_layers, x_p, pg_ei, pg_ea, pg_layers, zrows):
    m_src, m_dst = mg_ei[0], mg_ei[1]
    p_src, p_dst = pg_ei[0], pg_ei[1]
    for lm, lp in zip(mg_layers, pg_layers):
        (mWl, mWr, mWe, matt, mb, mlW, mlb) = lm
        (pWl, pWr, pWe, patt, pb, plW, plb) = lp
        m_xl, m_xr = _mm2(x_m, mWl, mWr)
        p_xl, p_xr = _mm2(x_p, pWl, pWr)
        m_XS, m_XD = _sc_gather2(m_xl, m_xr, m_src, m_dst)
        p_XS, p_XD = _sc_gather2(p_xl, p_xr, p_src, p_dst)
        m_R0, m_R1 = _edge(m_XS, m_XD, mg_ea, mWe, matt)
        p_R0, p_R1 = _edge(p_XS, p_XD, pg_ea, pWe, patt)
        m_a0, m_a1 = _sc_scatter(m_R0, m_R1, m_dst, zrows)
        p_a0, p_a1 = _sc_scatter(p_R0, p_R1, p_dst, zrows)
        x_m = _finish(m_a0, m_a1, mb, mlW, mlb)
        x_p = _finish(p_a0, p_a1, pb, plW, plb)
    return x_m, x_p


def kernel(mg_x, mg_edge_index, mg_edge_attr, pg_x, pg_edge_index, pg_edge_attr,
           mg_Wl0, mg_Wr0, mg_We0, mg_att0, mg_b0, mg_lW0, mg_lb0,
           mg_Wl1, mg_Wr1, mg_We1, mg_att1, mg_b1, mg_lW1, mg_lb1,
           pg_Wl0, pg_Wr0, pg_We0, pg_att0, pg_b0, pg_lW0, pg_lb0,
           pg_Wl1, pg_Wr1, pg_We1, pg_att1, pg_b1, pg_lW1, pg_lb1,
           f1W, f1b, f2W, f2b):
    mg_layers = [
        (mg_Wl0, mg_Wr0, mg_We0, mg_att0, mg_b0, mg_lW0, mg_lb0),
        (mg_Wl1, mg_Wr1, mg_We1, mg_att1, mg_b1, mg_lW1, mg_lb1),
    ]
    pg_layers = [
        (pg_Wl0, pg_Wr0, pg_We0, pg_att0, pg_b0, pg_lW0, pg_lb0),
        (pg_Wl1, pg_Wr1, pg_We1, pg_att1, pg_b1, pg_lW1, pg_lb1),
    ]
    zrows = jnp.zeros((_STR, ACC_W), jnp.float32)
    x_m, x_p = _two_blocks(mg_x, mg_edge_index, mg_edge_attr, mg_layers,
                           pg_x, pg_edge_index, pg_edge_attr, pg_layers, zrows)
    return _head(x_m, x_p, f1W, f1b, f2W, f2b)


# speedup vs baseline: 1.1028x; 1.0196x over previous
import functools
import numpy as np
import jax
import jax.numpy as jnp
from jax import lax
from jax.experimental import pallas as pl
from jax.experimental.pallas import tpu as pltpu
from jax.experimental.pallas import tpu_sc as plsc

N_NODES = 10000
E_EDGES = 160000
OUT, HEADS, CPH = 256, 8, 32
D_EDGE = 16
ACC_W = 144

_NC, _NS = 2, 16
_NW = _NC * _NS
_GC = 200
_EW = E_EDGES // _NW
_SCC = 200
_ET = E_EDGES // _NS
_STR = 640
_STR_LAST = N_NODES - 15 * _STR

_BLK = np.zeros((OUT, HEADS), np.float32)
for _h in range(HEADS):
    _BLK[_h * CPH:(_h + 1) * CPH, _h] = 1.0
_BLKT = _BLK.T.copy()
_P0 = np.zeros((HEADS, 16), np.float32)
_P1 = np.zeros((HEADS, 16), np.float32)
for _h in range(4):
    _P0[_h, _h] = 1.0
    _P1[_h + 4, _h] = 1.0
_B16 = np.zeros((16, 128), np.float32)
for _h in range(4):
    _B16[_h, _h * CPH:(_h + 1) * CPH] = 1.0


def _pack_bf16_pair(y):
    lo = lax.bitcast_convert_type(y[:, :128].astype(jnp.bfloat16), jnp.uint16)
    hi = lax.bitcast_convert_type(y[:, 128:].astype(jnp.bfloat16), jnp.uint16)
    word = lo.astype(jnp.uint32) | (hi.astype(jnp.uint32) << 16)
    return lax.bitcast_convert_type(word, jnp.float32)


def _unpack_bf16_pair(p):
    u = lax.bitcast_convert_type(p, jnp.uint32)
    lo = lax.bitcast_convert_type((u & 0xFFFF).astype(jnp.uint16), jnp.bfloat16)
    hi = lax.bitcast_convert_type((u >> 16).astype(jnp.uint16), jnp.bfloat16)
    return lo.astype(jnp.float32), hi.astype(jnp.float32)


def _mm2_body(x_ref, wl_ref, wr_ref, ol_ref, or_ref):
    x = x_ref[...]
    ol_ref[...] = _pack_bf16_pair(jnp.dot(x, wl_ref[...], preferred_element_type=jnp.float32))
    or_ref[...] = _pack_bf16_pair(jnp.dot(x, wr_ref[...], preferred_element_type=jnp.float32))


def _mm2(x, Wl, Wr):
    M, K = x.shape
    BM = 400
    return pl.pallas_call(
        _mm2_body,
        grid=(M // BM,),
        in_specs=[
            pl.BlockSpec((BM, K), lambda i: (i, 0)),
            pl.BlockSpec((K, OUT), lambda i: (0, 0)),
            pl.BlockSpec((K, OUT), lambda i: (0, 0)),
        ],
        out_specs=[
            pl.BlockSpec((BM, 128), lambda i: (i, 0)),
            pl.BlockSpec((BM, 128), lambda i: (i, 0)),
        ],
        out_shape=[
            jax.ShapeDtypeStruct((M, 128), jnp.float32),
            jax.ShapeDtypeStruct((M, 128), jnp.float32),
        ],
    )(x, Wl, Wr)


def _edge_body(xs_ref, xd_ref, ea_ref, we_ref, a_ref, bt_ref, p0_ref, p1_ref,
               r0_ref, r1_ref):
    xs0, xs1 = _unpack_bf16_pair(xs_ref[...])
    xd0, xd1 = _unpack_bf16_pair(xd_ref[...])
    e = jnp.dot(ea_ref[...], we_ref[...], preferred_element_type=jnp.float32)
    m0 = xs0 + xd0 + e[:, :128]
    m1 = xs1 + xd1 + e[:, 128:]
    m0 = jnp.where(m0 >= 0, m0, 0.2 * m0)
    m1 = jnp.where(m1 >= 0, m1, 0.2 * m1)
    alpha = (jnp.dot(m0, a_ref[:128, :], preferred_element_type=jnp.float32)
             + jnp.dot(m1, a_ref[128:, :], preferred_element_type=jnp.float32))
    w = jnp.exp(alpha)
    wb0 = jnp.dot(w, bt_ref[:, :128], preferred_element_type=jnp.float32)
    wb1 = jnp.dot(w, bt_ref[:, 128:], preferred_element_type=jnp.float32)
    w0 = jnp.dot(w, p0_ref[...], preferred_element_type=jnp.float32)
    w1 = jnp.dot(w, p1_ref[...], preferred_element_type=jnp.float32)
    r0_ref[...] = jnp.concatenate([xs0 * wb0, w0], axis=1)
    r1_ref[...] = jnp.concatenate([xs1 * wb1, w1], axis=1)


def _edge(XS, XD, ea, We, att):
    BE = 4000
    A = att.reshape(-1)[:, None] * jnp.asarray(_BLK)
    return pl.pallas_call(
        _edge_body,
        grid=(E_EDGES // BE,),
        in_specs=[
            pl.BlockSpec((BE, 128), lambda i: (i, 0)),
            pl.BlockSpec((BE, 128), lambda i: (i, 0)),
            pl.BlockSpec((BE, D_EDGE), lambda i: (i, 0)),
            pl.BlockSpec((D_EDGE, OUT), lambda i: (0, 0)),
            pl.BlockSpec((OUT, HEADS), lambda i: (0, 0)),
            pl.BlockSpec((HEADS, OUT), lambda i: (0, 0)),
            pl.BlockSpec((HEADS, 16), lambda i: (0, 0)),
            pl.BlockSpec((HEADS, 16), lambda i: (0, 0)),
        ],
        out_specs=[
            pl.BlockSpec((BE, ACC_W), lambda i: (i, 0)),
            pl.BlockSpec((BE, ACC_W), lambda i: (i, 0)),
        ],
        out_shape=[
            jax.ShapeDtypeStruct((E_EDGES, ACC_W), jnp.float32),
            jax.ShapeDtypeStruct((E_EDGES, ACC_W), jnp.float32),
        ],
    )(XS, XD, ea, We, A, jnp.asarray(_BLKT), jnp.asarray(_P0), jnp.asarray(_P1))


def _finish_body(a0_ref, a1_ref, b16_ref, b_ref, lw_ref, lb_ref, o_ref):
    db0 = jnp.dot(a0_ref[:, 128:144], b16_ref[...], preferred_element_type=jnp.float32)
    db1 = jnp.dot(a1_ref[:, 128:144], b16_ref[...], preferred_element_type=jnp.float32)
    x0 = a0_ref[:, 0:128] / (db0 + 1e-16)
    x1 = a1_ref[:, 0:128] / (db1 + 1e-16)
    x = jnp.concatenate([x0, x1], axis=1) + b_ref[...]
    y = jnp.dot(x, lw_ref[...], preferred_element_type=jnp.float32) + lb_ref[...]
    o_ref[...] = x + jnp.where(y > 0, y, jnp.exp(jnp.minimum(y, 0.0)) - 1.0)


def _finish(acc0, acc1, bias, lW, lb):
    BM = 400
    return pl.pallas_call(
        _finish_body,
        grid=(N_NODES // BM,),
        in_specs=[
            pl.BlockSpec((BM, ACC_W), lambda i: (i, 0)),
            pl.BlockSpec((BM, ACC_W), lambda i: (i, 0)),
            pl.BlockSpec((16, 128), lambda i: (0, 0)),
            pl.BlockSpec((1, OUT), lambda i: (0, 0)),
            pl.BlockSpec((OUT, OUT), lambda i: (0, 0)),
            pl.BlockSpec((1, OUT), lambda i: (0, 0)),
        ],
        out_specs=pl.BlockSpec((BM, OUT), lambda i: (i, 0)),
        out_shape=jax.ShapeDtypeStruct((N_NODES, OUT), jnp.float32),
    )(acc0, acc1, jnp.asarray(_B16), bias.reshape(1, OUT), lW, lb.reshape(1, OUT))


def _head_body(xm_ref, xp_ref, f1m_ref, f1p_ref, f1b_ref, f2w_ref, f2b_ref,
               o_ref, sm_ref, sp_ref):
    i = pl.program_id(0)

    @pl.when(i == 0)
    def _():
        sm_ref[...] = jnp.zeros_like(sm_ref)
        sp_ref[...] = jnp.zeros_like(sp_ref)

    sm_ref[...] += jnp.sum(xm_ref[...], 0, keepdims=True)
    sp_ref[...] += jnp.sum(xp_ref[...], 0, keepdims=True)

    @pl.when(i == pl.num_programs(0) - 1)
    def _():
        z = (jnp.dot(sm_ref[...], f1m_ref[...], preferred_element_type=jnp.float32)
             + jnp.dot(sp_ref[...], f1p_ref[...], preferred_element_type=jnp.float32)
             + f1b_ref[...])
        z = jnp.where(z > 0, z, jnp.exp(jnp.minimum(z, 0.0)) - 1.0)
        t = jnp.dot(z, f2w_ref[...], preferred_element_type=jnp.float32) + f2b_ref[...]
        o_ref[...] = 1.0 / (1.0 + jnp.exp(-t[0:1, 0:1]))


def _head(x_m, x_p, f1W, f1b, f2W, f2b):
    BM = 400
    return pl.pallas_call(
        _head_body,
        grid=(N_NODES // BM,),
        in_specs=[
            pl.BlockSpec((BM, OUT), lambda i: (i, 0)),
            pl.BlockSpec((BM, OUT), lambda i: (i, 0)),
            pl.BlockSpec((OUT, OUT), lambda i: (0, 0)),
            pl.BlockSpec((OUT, OUT), lambda i: (0, 0)),
            pl.BlockSpec((1, OUT), lambda i: (0, 0)),
            pl.BlockSpec((OUT, 1), lambda i: (0, 0)),
            pl.BlockSpec((1, 1), lambda i: (0, 0)),
        ],
        out_specs=pl.BlockSpec((1, 1), lambda i: (0, 0)),
        out_shape=jax.ShapeDtypeStruct((1, 1), jnp.float32),
        scratch_shapes=[
            pltpu.VMEM((1, OUT), jnp.float32),
            pltpu.VMEM((1, OUT), jnp.float32),
        ],
    )(x_m, x_p, f1W[:OUT], f1W[OUT:], f1b.reshape(1, OUT), f2W, f2b.reshape(1, 1))


def _sc_gather2(xl, xr, src, dst):
    mesh = plsc.VectorSubcoreMesh(core_axis_name="c", subcore_axis_name="s")

    @functools.partial(
        pl.kernel,
        out_type=[jax.ShapeDtypeStruct((E_EDGES, 128), jnp.float32),
                  jax.ShapeDtypeStruct((E_EDGES, 128), jnp.float32)],
        mesh=mesh,
        scratch_types=[
            pltpu.VMEM((_GC,), jnp.int32),
            pltpu.VMEM((_GC,), jnp.int32),
            pltpu.VMEM((_GC, 128), jnp.float32),
            pltpu.VMEM((_GC, 128), jnp.float32),
            pltpu.SemaphoreType.DMA,
            pltpu.SemaphoreType.DMA,
        ],
    )
    def k(xl_hbm, xr_hbm, src_hbm, dst_hbm, xs_hbm, xd_hbm,
          sidx, didx, srows, drows, sem1, sem2):
        wid = lax.axis_index("s") * _NC + lax.axis_index("c")
        w0 = wid * _EW

        def body(i, _):
            base = w0 + i * _GC
            pltpu.sync_copy(src_hbm.at[pl.ds(base, _GC)], sidx)
            pltpu.sync_copy(dst_hbm.at[pl.ds(base, _GC)], didx)
            a = pltpu.async_copy(xl_hbm.at[sidx], srows, sem1)
            b = pltpu.async_copy(xr_hbm.at[didx], drows, sem2)
            a.wait()
            pltpu.sync_copy(srows, xs_hbm.at[pl.ds(base, _GC)])
            b.wait()
            pltpu.sync_copy(drows, xd_hbm.at[pl.ds(base, _GC)])
            return 0

        lax.fori_loop(0, _EW // _GC, body, 0)

    return k(xl, xr, src, dst)


def _sc_scatter(R0, R1, dst, zrows):
    mesh = plsc.VectorSubcoreMesh(core_axis_name="c", subcore_axis_name="s")

    @functools.partial(
        pl.kernel,
        out_type=[jax.ShapeDtypeStruct((N_NODES, ACC_W), jnp.float32),
                  jax.ShapeDtypeStruct((N_NODES, ACC_W), jnp.float32)],
        mesh=mesh,
        scratch_types=[
            pltpu.VMEM((_SCC,), jnp.int32),
            pltpu.VMEM((_SCC, ACC_W), jnp.float32),
            pltpu.VMEM_SHARED((N_NODES, ACC_W), jnp.float32),
            pltpu.SemaphoreType.DMA,
        ],
        compiler_params=pltpu.CompilerParams(use_tc_tiling_on_sc=False),
    )
    def k(r0_hbm, r1_hbm, dst_hbm, z_hbm, a0_hbm, a1_hbm, idxb, valb, acc, sem):
        sc = lax.axis_index("c")
        t = lax.axis_index("s")

        @pl.when(t < 15)
        def _():
            pltpu.sync_copy(z_hbm, acc.at[pl.ds(t * _STR, _STR)])

        @pl.when(t == 15)
        def _():
            pltpu.sync_copy(z_hbm.at[pl.ds(0, _STR_LAST)],
                            acc.at[pl.ds(15 * _STR, _STR_LAST)])

        plsc.subcore_barrier()

        def body(i, _):
            base = t * _ET + i * _SCC
            pltpu.sync_copy(dst_hbm.at[pl.ds(base, _SCC)], idxb)

            @pl.when(sc == 0)
            def _():
                pltpu.sync_copy(r0_hbm.at[pl.ds(base, _SCC)], valb)

            @pl.when(sc == 1)
            def _():
                pltpu.sync_copy(r1_hbm.at[pl.ds(base, _SCC)], valb)

            pltpu.sync_copy(valb, acc.at[idxb], add=True)
            return 0

        lax.fori_loop(0, _ET // _SCC, body, 0)
        plsc.subcore_barrier()

        def wb(out_hbm):
            @pl.when(t < 15)
            def _():
                pltpu.sync_copy(acc.at[pl.ds(t * _STR, _STR)],
                                out_hbm.at[pl.ds(t * _STR, _STR)])

            @pl.when(t == 15)
            def _():
                pltpu.sync_copy(acc.at[pl.ds(15 * _STR, _STR_LAST)],
                                out_hbm.at[pl.ds(15 * _STR, _STR_LAST)])

        @pl.when(sc == 0)
        def _():
            wb(a0_hbm)

        @pl.when(sc == 1)
        def _():
            wb(a1_hbm)

    return k(R0, R1, dst, zrows)


def _gat_layer(x, src, dst, ea, zrows, Wl, Wr, We, att, bias, lW, lb):
    xl, xr = _mm2(x, Wl, Wr)
    XS, XD = _sc_gather2(xl, xr, src, dst)
    R0, R1 = _edge(XS, XD, ea, We, att)
    acc0, acc1 = _sc_scatter(R0, R1, dst, zrows)
    return _finish(acc0, acc1, bias, lW, lb)


def _two_blocks(x_m, mg_ei, mg_ea, mg_layers, x_p, pg_ei, pg_ea, pg_layers, zrows):
    m_src, m_dst = mg_ei[0], mg_ei[1]
    p_src, p_dst = pg_ei[0], pg_ei[1]
    for lm, lp in zip(mg_layers, pg_layers):
        (mWl, mWr, mWe, matt, mb, mlW, mlb) = lm
        (pWl, pWr, pWe, patt, pb, plW, plb) = lp
        m_xl, m_xr = _mm2(x_m, mWl, mWr)
        p_xl, p_xr = _mm2(x_p, pWl, pWr)
        m_XS, m_XD = _sc_gather2(m_xl, m_xr, m_src, m_dst)
        p_XS, p_XD = _sc_gather2(p_xl, p_xr, p_src, p_dst)
        m_R0, m_R1 = _edge(m_XS, m_XD, mg_ea, mWe, matt)
        p_R0, p_R1 = _edge(p_XS, p_XD, pg_ea, pWe, patt)
        m_a0, m_a1 = _sc_scatter(m_R0, m_R1, m_dst, zrows)
        p_a0, p_a1 = _sc_scatter(p_R0, p_R1, p_dst, zrows)
        x_m = _finish(m_a0, m_a1, mb, mlW, mlb)
        x_p = _finish(p_a0, p_a1, pb, plW, plb)
    return x_m, x_p


def kernel(mg_x, mg_edge_index, mg_edge_attr, pg_x, pg_edge_index, pg_edge_attr,
           mg_Wl0, mg_Wr0, mg_We0, mg_att0, mg_b0, mg_lW0, mg_lb0,
           mg_Wl1, mg_Wr1, mg_We1, mg_att1, mg_b1, mg_lW1, mg_lb1,
           pg_Wl0, pg_Wr0, pg_We0, pg_att0, pg_b0, pg_lW0, pg_lb0,
           pg_Wl1, pg_Wr1, pg_We1, pg_att1, pg_b1, pg_lW1, pg_lb1,
           f1W, f1b, f2W, f2b):
    mg_layers = [
        (mg_Wl0, mg_Wr0, mg_We0, mg_att0, mg_b0, mg_lW0, mg_lb0),
        (mg_Wl1, mg_Wr1, mg_We1, mg_att1, mg_b1, mg_lW1, mg_lb1),
    ]
    pg_layers = [
        (pg_Wl0, pg_Wr0, pg_We0, pg_att0, pg_b0, pg_lW0, pg_lb0),
        (pg_Wl1, pg_Wr1, pg_We1, pg_att1, pg_b1, pg_lW1, pg_lb1),
    ]
    zrows = jnp.zeros((_STR, ACC_W), jnp.float32)
    x_m, x_p = _two_blocks(mg_x, mg_edge_index, mg_edge_attr, mg_layers,
                           pg_x, pg_edge_index, pg_edge_attr, pg_layers, zrows)
    return _head(x_m, x_p, f1W, f1b, f2W, f2b)


# speedup vs baseline: 1.1106x; 1.0071x over previous
import functools
import numpy as np
import jax
import jax.numpy as jnp
from jax import lax
from jax.experimental import pallas as pl
from jax.experimental.pallas import tpu as pltpu
from jax.experimental.pallas import tpu_sc as plsc

N_NODES = 10000
E_EDGES = 160000
OUT, HEADS, CPH = 256, 8, 32
D_EDGE = 16
ACC_W = 144

_NC, _NS = 2, 16
_NW = _NC * _NS
_GC = 200
_EW = E_EDGES // _NW
_SCC = 200
_ET = E_EDGES // _NS
_STR = 640
_STR_LAST = N_NODES - 15 * _STR

_BLK = np.zeros((OUT, HEADS), np.float32)
for _h in range(HEADS):
    _BLK[_h * CPH:(_h + 1) * CPH, _h] = 1.0
_BLKT = _BLK.T.copy()
_P0 = np.zeros((HEADS, 16), np.float32)
_P1 = np.zeros((HEADS, 16), np.float32)
for _h in range(4):
    _P0[_h, _h] = 1.0
    _P1[_h + 4, _h] = 1.0
_B16 = np.zeros((16, 128), np.float32)
for _h in range(4):
    _B16[_h, _h * CPH:(_h + 1) * CPH] = 1.0


def _pack_bf16_pair(y):
    lo = lax.bitcast_convert_type(y[:, :128].astype(jnp.bfloat16), jnp.uint16)
    hi = lax.bitcast_convert_type(y[:, 128:].astype(jnp.bfloat16), jnp.uint16)
    word = lo.astype(jnp.uint32) | (hi.astype(jnp.uint32) << 16)
    return lax.bitcast_convert_type(word, jnp.float32)


def _unpack_bf16_pair(p):
    u = lax.bitcast_convert_type(p, jnp.uint32)
    lo = lax.bitcast_convert_type((u & 0xFFFF).astype(jnp.uint16), jnp.bfloat16)
    hi = lax.bitcast_convert_type((u >> 16).astype(jnp.uint16), jnp.bfloat16)
    return lo.astype(jnp.float32), hi.astype(jnp.float32)


def _mm2_body(x_ref, wl_ref, wr_ref, ol_ref, or_ref):
    x = x_ref[...]
    ol_ref[...] = _pack_bf16_pair(jnp.dot(x, wl_ref[...], preferred_element_type=jnp.float32))
    or_ref[...] = _pack_bf16_pair(jnp.dot(x, wr_ref[...], preferred_element_type=jnp.float32))


def _mm2(x, Wl, Wr):
    M, K = x.shape
    BM = 1000
    return pl.pallas_call(
        _mm2_body,
        grid=(M // BM,),
        in_specs=[
            pl.BlockSpec((BM, K), lambda i: (i, 0)),
            pl.BlockSpec((K, OUT), lambda i: (0, 0)),
            pl.BlockSpec((K, OUT), lambda i: (0, 0)),
        ],
        out_specs=[
            pl.BlockSpec((BM, 128), lambda i: (i, 0)),
            pl.BlockSpec((BM, 128), lambda i: (i, 0)),
        ],
        out_shape=[
            jax.ShapeDtypeStruct((M, 128), jnp.float32),
            jax.ShapeDtypeStruct((M, 128), jnp.float32),
        ],
    )(x, Wl, Wr)


def _edge_body(xs_ref, xd_ref, ea_ref, we_ref, a_ref, bt_ref, p0_ref, p1_ref,
               r0_ref, r1_ref):
    xs0, xs1 = _unpack_bf16_pair(xs_ref[...])
    xd0, xd1 = _unpack_bf16_pair(xd_ref[...])
    e = jnp.dot(ea_ref[...], we_ref[...], preferred_element_type=jnp.float32)
    m0 = xs0 + xd0 + e[:, :128]
    m1 = xs1 + xd1 + e[:, 128:]
    m0 = jnp.where(m0 >= 0, m0, 0.2 * m0)
    m1 = jnp.where(m1 >= 0, m1, 0.2 * m1)
    alpha = (jnp.dot(m0, a_ref[:128, :], preferred_element_type=jnp.float32)
             + jnp.dot(m1, a_ref[128:, :], preferred_element_type=jnp.float32))
    w = jnp.exp(alpha)
    wb0 = jnp.dot(w, bt_ref[:, :128], preferred_element_type=jnp.float32)
    wb1 = jnp.dot(w, bt_ref[:, 128:], preferred_element_type=jnp.float32)
    w0 = jnp.dot(w, p0_ref[...], preferred_element_type=jnp.float32)
    w1 = jnp.dot(w, p1_ref[...], preferred_element_type=jnp.float32)
    r0_ref[...] = jnp.concatenate([xs0 * wb0, w0], axis=1)
    r1_ref[...] = jnp.concatenate([xs1 * wb1, w1], axis=1)


def _edge(XS, XD, ea, We, att):
    BE = 4000
    A = att.reshape(-1)[:, None] * jnp.asarray(_BLK)
    return pl.pallas_call(
        _edge_body,
        grid=(E_EDGES // BE,),
        in_specs=[
            pl.BlockSpec((BE, 128), lambda i: (i, 0)),
            pl.BlockSpec((BE, 128), lambda i: (i, 0)),
            pl.BlockSpec((BE, D_EDGE), lambda i: (i, 0)),
            pl.BlockSpec((D_EDGE, OUT), lambda i: (0, 0)),
            pl.BlockSpec((OUT, HEADS), lambda i: (0, 0)),
            pl.BlockSpec((HEADS, OUT), lambda i: (0, 0)),
            pl.BlockSpec((HEADS, 16), lambda i: (0, 0)),
            pl.BlockSpec((HEADS, 16), lambda i: (0, 0)),
        ],
        out_specs=[
            pl.BlockSpec((BE, ACC_W), lambda i: (i, 0)),
            pl.BlockSpec((BE, ACC_W), lambda i: (i, 0)),
        ],
        out_shape=[
            jax.ShapeDtypeStruct((E_EDGES, ACC_W), jnp.float32),
            jax.ShapeDtypeStruct((E_EDGES, ACC_W), jnp.float32),
        ],
    )(XS, XD, ea, We, A, jnp.asarray(_BLKT), jnp.asarray(_P0), jnp.asarray(_P1))


def _finish_body(a0_ref, a1_ref, b16_ref, b_ref, lw_ref, lb_ref, o_ref):
    db0 = jnp.dot(a0_ref[:, 128:144], b16_ref[...], preferred_element_type=jnp.float32)
    db1 = jnp.dot(a1_ref[:, 128:144], b16_ref[...], preferred_element_type=jnp.float32)
    x0 = a0_ref[:, 0:128] / (db0 + 1e-16)
    x1 = a1_ref[:, 0:128] / (db1 + 1e-16)
    x = jnp.concatenate([x0, x1], axis=1) + b_ref[...]
    y = jnp.dot(x, lw_ref[...], preferred_element_type=jnp.float32) + lb_ref[...]
    o_ref[...] = x + jnp.where(y > 0, y, jnp.exp(jnp.minimum(y, 0.0)) - 1.0)


def _finish(acc0, acc1, bias, lW, lb):
    BM = 1000
    return pl.pallas_call(
        _finish_body,
        grid=(N_NODES // BM,),
        in_specs=[
            pl.BlockSpec((BM, ACC_W), lambda i: (i, 0)),
            pl.BlockSpec((BM, ACC_W), lambda i: (i, 0)),
            pl.BlockSpec((16, 128), lambda i: (0, 0)),
            pl.BlockSpec((1, OUT), lambda i: (0, 0)),
            pl.BlockSpec((OUT, OUT), lambda i: (0, 0)),
            pl.BlockSpec((1, OUT), lambda i: (0, 0)),
        ],
        out_specs=pl.BlockSpec((BM, OUT), lambda i: (i, 0)),
        out_shape=jax.ShapeDtypeStruct((N_NODES, OUT), jnp.float32),
    )(acc0, acc1, jnp.asarray(_B16), bias.reshape(1, OUT), lW, lb.reshape(1, OUT))


def _head_body(xm_ref, xp_ref, f1m_ref, f1p_ref, f1b_ref, f2w_ref, f2b_ref,
               o_ref, sm_ref, sp_ref):
    i = pl.program_id(0)

    @pl.when(i == 0)
    def _():
        sm_ref[...] = jnp.zeros_like(sm_ref)
        sp_ref[...] = jnp.zeros_like(sp_ref)

    sm_ref[...] += jnp.sum(xm_ref[...], 0, keepdims=True)
    sp_ref[...] += jnp.sum(xp_ref[...], 0, keepdims=True)

    @pl.when(i == pl.num_programs(0) - 1)
    def _():
        z = (jnp.dot(sm_ref[...], f1m_ref[...], preferred_element_type=jnp.float32)
             + jnp.dot(sp_ref[...], f1p_ref[...], preferred_element_type=jnp.float32)
             + f1b_ref[...])
        z = jnp.where(z > 0, z, jnp.exp(jnp.minimum(z, 0.0)) - 1.0)
        t = jnp.dot(z, f2w_ref[...], preferred_element_type=jnp.float32) + f2b_ref[...]
        o_ref[...] = 1.0 / (1.0 + jnp.exp(-t[0:1, 0:1]))


def _head(x_m, x_p, f1W, f1b, f2W, f2b):
    BM = 400
    return pl.pallas_call(
        _head_body,
        grid=(N_NODES // BM,),
        in_specs=[
            pl.BlockSpec((BM, OUT), lambda i: (i, 0)),
            pl.BlockSpec((BM, OUT), lambda i: (i, 0)),
            pl.BlockSpec((OUT, OUT), lambda i: (0, 0)),
            pl.BlockSpec((OUT, OUT), lambda i: (0, 0)),
            pl.BlockSpec((1, OUT), lambda i: (0, 0)),
            pl.BlockSpec((OUT, 1), lambda i: (0, 0)),
            pl.BlockSpec((1, 1), lambda i: (0, 0)),
        ],
        out_specs=pl.BlockSpec((1, 1), lambda i: (0, 0)),
        out_shape=jax.ShapeDtypeStruct((1, 1), jnp.float32),
        scratch_shapes=[
            pltpu.VMEM((1, OUT), jnp.float32),
            pltpu.VMEM((1, OUT), jnp.float32),
        ],
    )(x_m, x_p, f1W[:OUT], f1W[OUT:], f1b.reshape(1, OUT), f2W, f2b.reshape(1, 1))


def _sc_gather2(xl, xr, src, dst):
    mesh = plsc.VectorSubcoreMesh(core_axis_name="c", subcore_axis_name="s")

    @functools.partial(
        pl.kernel,
        out_type=[jax.ShapeDtypeStruct((E_EDGES, 128), jnp.float32),
                  jax.ShapeDtypeStruct((E_EDGES, 128), jnp.float32)],
        mesh=mesh,
        scratch_types=[
            pltpu.VMEM((_GC,), jnp.int32),
            pltpu.VMEM((_GC,), jnp.int32),
            pltpu.VMEM((_GC, 128), jnp.float32),
            pltpu.VMEM((_GC, 128), jnp.float32),
            pltpu.SemaphoreType.DMA,
            pltpu.SemaphoreType.DMA,
        ],
    )
    def k(xl_hbm, xr_hbm, src_hbm, dst_hbm, xs_hbm, xd_hbm,
          sidx, didx, srows, drows, sem1, sem2):
        wid = lax.axis_index("s") * _NC + lax.axis_index("c")
        w0 = wid * _EW

        def body(i, _):
            base = w0 + i * _GC
            pltpu.sync_copy(src_hbm.at[pl.ds(base, _GC)], sidx)
            pltpu.sync_copy(dst_hbm.at[pl.ds(base, _GC)], didx)
            a = pltpu.async_copy(xl_hbm.at[sidx], srows, sem1)
            b = pltpu.async_copy(xr_hbm.at[didx], drows, sem2)
            a.wait()
            pltpu.sync_copy(srows, xs_hbm.at[pl.ds(base, _GC)])
            b.wait()
            pltpu.sync_copy(drows, xd_hbm.at[pl.ds(base, _GC)])
            return 0

        lax.fori_loop(0, _EW // _GC, body, 0)

    return k(xl, xr, src, dst)


def _sc_scatter(R0, R1, dst, zrows):
    mesh = plsc.VectorSubcoreMesh(core_axis_name="c", subcore_axis_name="s")

    @functools.partial(
        pl.kernel,
        out_type=[jax.ShapeDtypeStruct((N_NODES, ACC_W), jnp.float32),
                  jax.ShapeDtypeStruct((N_NODES, ACC_W), jnp.float32)],
        mesh=mesh,
        scratch_types=[
            pltpu.VMEM((_SCC,), jnp.int32),
            pltpu.VMEM((_SCC, ACC_W), jnp.float32),
            pltpu.VMEM_SHARED((N_NODES, ACC_W), jnp.float32),
            pltpu.SemaphoreType.DMA,
        ],
        compiler_params=pltpu.CompilerParams(use_tc_tiling_on_sc=False),
    )
    def k(r0_hbm, r1_hbm, dst_hbm, z_hbm, a0_hbm, a1_hbm, idxb, valb, acc, sem):
        sc = lax.axis_index("c")
        t = lax.axis_index("s")

        @pl.when(t < 15)
        def _():
            pltpu.sync_copy(z_hbm, acc.at[pl.ds(t * _STR, _STR)])

        @pl.when(t == 15)
        def _():
            pltpu.sync_copy(z_hbm.at[pl.ds(0, _STR_LAST)],
                            acc.at[pl.ds(15 * _STR, _STR_LAST)])

        plsc.subcore_barrier()

        def body(i, _):
            base = t * _ET + i * _SCC
            pltpu.sync_copy(dst_hbm.at[pl.ds(base, _SCC)], idxb)

            @pl.when(sc == 0)
            def _():
                pltpu.sync_copy(r0_hbm.at[pl.ds(base, _SCC)], valb)

            @pl.when(sc == 1)
            def _():
                pltpu.sync_copy(r1_hbm.at[pl.ds(base, _SCC)], valb)

            pltpu.sync_copy(valb, acc.at[idxb], add=True)
            return 0

        lax.fori_loop(0, _ET // _SCC, body, 0)
        plsc.subcore_barrier()

        def wb(out_hbm):
            @pl.when(t < 15)
            def _():
                pltpu.sync_copy(acc.at[pl.ds(t * _STR, _STR)],
                                out_hbm.at[pl.ds(t * _STR, _STR)])

            @pl.when(t == 15)
            def _():
                pltpu.sync_copy(acc.at[pl.ds(15 * _STR, _STR_LAST)],
                                out_hbm.at[pl.ds(15 * _STR, _STR_LAST)])

        @pl.when(sc == 0)
        def _():
            wb(a0_hbm)

        @pl.when(sc == 1)
        def _():
            wb(a1_hbm)

    return k(R0, R1, dst, zrows)


def _gat_layer(x, src, dst, ea, zrows, Wl, Wr, We, att, bias, lW, lb):
    xl, xr = _mm2(x, Wl, Wr)
    XS, XD = _sc_gather2(xl, xr, src, dst)
    R0, R1 = _edge(XS, XD, ea, We, att)
    acc0, acc1 = _sc_scatter(R0, R1, dst, zrows)
    return _finish(acc0, acc1, bias, lW, lb)


def _two_blocks(x_m, mg_ei, mg_ea, mg_layers, x_p, pg_ei, pg_ea, pg_layers, zrows):
    m_src, m_dst = mg_ei[0], mg_ei[1]
    p_src, p_dst = pg_ei[0], pg_ei[1]
    for lm, lp in zip(mg_layers, pg_layers):
        (mWl, mWr, mWe, matt, mb, mlW, mlb) = lm
        (pWl, pWr, pWe, patt, pb, plW, plb) = lp
        m_xl, m_xr = _mm2(x_m, mWl, mWr)
        p_xl, p_xr = _mm2(x_p, pWl, pWr)
        m_XS, m_XD = _sc_gather2(m_xl, m_xr, m_src, m_dst)
        p_XS, p_XD = _sc_gather2(p_xl, p_xr, p_src, p_dst)
        m_R0, m_R1 = _edge(m_XS, m_XD, mg_ea, mWe, matt)
        p_R0, p_R1 = _edge(p_XS, p_XD, pg_ea, pWe, patt)
        m_a0, m_a1 = _sc_scatter(m_R0, m_R1, m_dst, zrows)
        p_a0, p_a1 = _sc_scatter(p_R0, p_R1, p_dst, zrows)
        x_m = _finish(m_a0, m_a1, mb, mlW, mlb)
        x_p = _finish(p_a0, p_a1, pb, plW, plb)
    return x_m, x_p


def kernel(mg_x, mg_edge_index, mg_edge_attr, pg_x, pg_edge_index, pg_edge_attr,
           mg_Wl0, mg_Wr0, mg_We0, mg_att0, mg_b0, mg_lW0, mg_lb0,
           mg_Wl1, mg_Wr1, mg_We1, mg_att1, mg_b1, mg_lW1, mg_lb1,
           pg_Wl0, pg_Wr0, pg_We0, pg_att0, pg_b0, pg_lW0, pg_lb0,
           pg_Wl1, pg_Wr1, pg_We1, pg_att1, pg_b1, pg_lW1, pg_lb1,
           f1W, f1b, f2W, f2b):
    mg_layers = [
        (mg_Wl0, mg_Wr0, mg_We0, mg_att0, mg_b0, mg_lW0, mg_lb0),
        (mg_Wl1, mg_Wr1, mg_We1, mg_att1, mg_b1, mg_lW1, mg_lb1),
    ]
    pg_layers = [
        (pg_Wl0, pg_Wr0, pg_We0, pg_att0, pg_b0, pg_lW0, pg_lb0),
        (pg_Wl1, pg_Wr1, pg_We1, pg_att1, pg_b1, pg_lW1, pg_lb1),
    ]
    zrows = jnp.zeros((_STR, ACC_W), jnp.float32)
    x_m, x_p = _two_blocks(mg_x, mg_edge_index, mg_edge_attr, mg_layers,
                           pg_x, pg_edge_index, pg_edge_attr, pg_layers, zrows)
    return _head(x_m, x_p, f1W, f1b, f2W, f2b)


# speedup vs baseline: 1.1209x; 1.0092x over previous
import functools
import numpy as np
import jax
import jax.numpy as jnp
from jax import lax
from jax.experimental import pallas as pl
from jax.experimental.pallas import tpu as pltpu
from jax.experimental.pallas import tpu_sc as plsc

N_NODES = 10000
E_EDGES = 160000
OUT, HEADS, CPH = 256, 8, 32
D_EDGE = 16
ACC_W = 144

_NC, _NS = 2, 16
_NW = _NC * _NS
_GC = 200
_EW = E_EDGES // _NW
_SCC = 200
_ET = E_EDGES // _NS
_STR = 640
_STR_LAST = N_NODES - 15 * _STR

_BLK = np.zeros((OUT, HEADS), np.float32)
for _h in range(HEADS):
    _BLK[_h * CPH:(_h + 1) * CPH, _h] = 1.0
_BLKT = _BLK.T.copy()
_P0 = np.zeros((HEADS, 16), np.float32)
_P1 = np.zeros((HEADS, 16), np.float32)
for _h in range(4):
    _P0[_h, _h] = 1.0
    _P1[_h + 4, _h] = 1.0
_B16 = np.zeros((16, 128), np.float32)
for _h in range(4):
    _B16[_h, _h * CPH:(_h + 1) * CPH] = 1.0


def _pack_bf16_pair(y):
    lo = lax.bitcast_convert_type(y[:, :128].astype(jnp.bfloat16), jnp.uint16)
    hi = lax.bitcast_convert_type(y[:, 128:].astype(jnp.bfloat16), jnp.uint16)
    word = lo.astype(jnp.uint32) | (hi.astype(jnp.uint32) << 16)
    return lax.bitcast_convert_type(word, jnp.float32)


def _unpack_bf16_pair(p):
    u = lax.bitcast_convert_type(p, jnp.uint32)
    lo = lax.bitcast_convert_type((u & 0xFFFF).astype(jnp.uint16), jnp.bfloat16)
    hi = lax.bitcast_convert_type((u >> 16).astype(jnp.uint16), jnp.bfloat16)
    return lo.astype(jnp.float32), hi.astype(jnp.float32)


def _mm2_body(x_ref, wl_ref, wr_ref, ol_ref, or_ref):
    x = x_ref[...]
    ol_ref[...] = _pack_bf16_pair(jnp.dot(x, wl_ref[...], preferred_element_type=jnp.float32))
    or_ref[...] = _pack_bf16_pair(jnp.dot(x, wr_ref[...], preferred_element_type=jnp.float32))


def _mm2(x, Wl, Wr):
    M, K = x.shape
    BM = 1000
    return pl.pallas_call(
        _mm2_body,
        grid=(M // BM,),
        in_specs=[
            pl.BlockSpec((BM, K), lambda i: (i, 0)),
            pl.BlockSpec((K, OUT), lambda i: (0, 0)),
            pl.BlockSpec((K, OUT), lambda i: (0, 0)),
        ],
        out_specs=[
            pl.BlockSpec((BM, 128), lambda i: (i, 0)),
            pl.BlockSpec((BM, 128), lambda i: (i, 0)),
        ],
        out_shape=[
            jax.ShapeDtypeStruct((M, 128), jnp.float32),
            jax.ShapeDtypeStruct((M, 128), jnp.float32),
        ],
    )(x, Wl, Wr)


def _edge_body(xs_ref, xd_ref, ea_ref, we_ref, a_ref, bt_ref, p0_ref, p1_ref,
               r0_ref, r1_ref):
    xs0, xs1 = _unpack_bf16_pair(xs_ref[...])
    xd0, xd1 = _unpack_bf16_pair(xd_ref[...])
    e = jnp.dot(ea_ref[...], we_ref[...], preferred_element_type=jnp.float32)
    m0 = xs0 + xd0 + e[:, :128]
    m1 = xs1 + xd1 + e[:, 128:]
    m0 = jnp.where(m0 >= 0, m0, 0.2 * m0)
    m1 = jnp.where(m1 >= 0, m1, 0.2 * m1)
    alpha = (jnp.dot(m0, a_ref[:128, :], preferred_element_type=jnp.float32)
             + jnp.dot(m1, a_ref[128:, :], preferred_element_type=jnp.float32))
    w = jnp.exp(alpha)
    wb0 = jnp.dot(w, bt_ref[:, :128], preferred_element_type=jnp.float32)
    wb1 = jnp.dot(w, bt_ref[:, 128:], preferred_element_type=jnp.float32)
    w0 = jnp.dot(w, p0_ref[...], preferred_element_type=jnp.float32)
    w1 = jnp.dot(w, p1_ref[...], preferred_element_type=jnp.float32)
    r0_ref[...] = jnp.concatenate([xs0 * wb0, w0], axis=1)
    r1_ref[...] = jnp.concatenate([xs1 * wb1, w1], axis=1)


def _edge(XS, XD, ea, We, att):
    BE = 4000
    A = att.reshape(-1)[:, None] * jnp.asarray(_BLK)
    return pl.pallas_call(
        _edge_body,
        grid=(E_EDGES // BE,),
        in_specs=[
            pl.BlockSpec((BE, 128), lambda i: (i, 0)),
            pl.BlockSpec((BE, 128), lambda i: (i, 0)),
            pl.BlockSpec((BE, D_EDGE), lambda i: (i, 0)),
            pl.BlockSpec((D_EDGE, OUT), lambda i: (0, 0)),
            pl.BlockSpec((OUT, HEADS), lambda i: (0, 0)),
            pl.BlockSpec((HEADS, OUT), lambda i: (0, 0)),
            pl.BlockSpec((HEADS, 16), lambda i: (0, 0)),
            pl.BlockSpec((HEADS, 16), lambda i: (0, 0)),
        ],
        out_specs=[
            pl.BlockSpec((BE, ACC_W), lambda i: (i, 0)),
            pl.BlockSpec((BE, ACC_W), lambda i: (i, 0)),
        ],
        out_shape=[
            jax.ShapeDtypeStruct((E_EDGES, ACC_W), jnp.float32),
            jax.ShapeDtypeStruct((E_EDGES, ACC_W), jnp.float32),
        ],
    )(XS, XD, ea, We, A, jnp.asarray(_BLKT), jnp.asarray(_P0), jnp.asarray(_P1))


def _finish_body(a0_ref, a1_ref, b16_ref, b_ref, lw_ref, lb_ref, o_ref):
    db0 = jnp.dot(a0_ref[:, 128:144], b16_ref[...], preferred_element_type=jnp.float32)
    db1 = jnp.dot(a1_ref[:, 128:144], b16_ref[...], preferred_element_type=jnp.float32)
    x0 = a0_ref[:, 0:128] / (db0 + 1e-16)
    x1 = a1_ref[:, 0:128] / (db1 + 1e-16)
    x = jnp.concatenate([x0, x1], axis=1) + b_ref[...]
    y = jnp.dot(x, lw_ref[...], preferred_element_type=jnp.float32) + lb_ref[...]
    o_ref[...] = x + jnp.where(y > 0, y, jnp.exp(jnp.minimum(y, 0.0)) - 1.0)


def _finish(acc0, acc1, bias, lW, lb):
    BM = 1000
    return pl.pallas_call(
        _finish_body,
        grid=(N_NODES // BM,),
        in_specs=[
            pl.BlockSpec((BM, ACC_W), lambda i: (i, 0)),
            pl.BlockSpec((BM, ACC_W), lambda i: (i, 0)),
            pl.BlockSpec((16, 128), lambda i: (0, 0)),
            pl.BlockSpec((1, OUT), lambda i: (0, 0)),
            pl.BlockSpec((OUT, OUT), lambda i: (0, 0)),
            pl.BlockSpec((1, OUT), lambda i: (0, 0)),
        ],
        out_specs=pl.BlockSpec((BM, OUT), lambda i: (i, 0)),
        out_shape=jax.ShapeDtypeStruct((N_NODES, OUT), jnp.float32),
    )(acc0, acc1, jnp.asarray(_B16), bias.reshape(1, OUT), lW, lb.reshape(1, OUT))


def _head_body(xm_ref, xp_ref, f1m_ref, f1p_ref, f1b_ref, f2w_ref, f2b_ref,
               o_ref, sm_ref, sp_ref):
    i = pl.program_id(0)

    @pl.when(i == 0)
    def _():
        sm_ref[...] = jnp.zeros_like(sm_ref)
        sp_ref[...] = jnp.zeros_like(sp_ref)

    sm_ref[...] += jnp.sum(xm_ref[...], 0, keepdims=True)
    sp_ref[...] += jnp.sum(xp_ref[...], 0, keepdims=True)

    @pl.when(i == pl.num_programs(0) - 1)
    def _():
        z = (jnp.dot(sm_ref[...], f1m_ref[...], preferred_element_type=jnp.float32)
             + jnp.dot(sp_ref[...], f1p_ref[...], preferred_element_type=jnp.float32)
             + f1b_ref[...])
        z = jnp.where(z > 0, z, jnp.exp(jnp.minimum(z, 0.0)) - 1.0)
        t = jnp.dot(z, f2w_ref[...], preferred_element_type=jnp.float32) + f2b_ref[...]
        o_ref[...] = 1.0 / (1.0 + jnp.exp(-t[0:1, 0:1]))


def _head(x_m, x_p, f1W, f1b, f2W, f2b):
    BM = 400
    return pl.pallas_call(
        _head_body,
        grid=(N_NODES // BM,),
        in_specs=[
            pl.BlockSpec((BM, OUT), lambda i: (i, 0)),
            pl.BlockSpec((BM, OUT), lambda i: (i, 0)),
            pl.BlockSpec((OUT, OUT), lambda i: (0, 0)),
            pl.BlockSpec((OUT, OUT), lambda i: (0, 0)),
            pl.BlockSpec((1, OUT), lambda i: (0, 0)),
            pl.BlockSpec((OUT, 1), lambda i: (0, 0)),
            pl.BlockSpec((1, 1), lambda i: (0, 0)),
        ],
        out_specs=pl.BlockSpec((1, 1), lambda i: (0, 0)),
        out_shape=jax.ShapeDtypeStruct((1, 1), jnp.float32),
        scratch_shapes=[
            pltpu.VMEM((1, OUT), jnp.float32),
            pltpu.VMEM((1, OUT), jnp.float32),
        ],
    )(x_m, x_p, f1W[:OUT], f1W[OUT:], f1b.reshape(1, OUT), f2W, f2b.reshape(1, 1))


def _sc_gather2(xl, xr, src, dst):
    mesh = plsc.VectorSubcoreMesh(core_axis_name="c", subcore_axis_name="s")

    @functools.partial(
        pl.kernel,
        out_type=[jax.ShapeDtypeStruct((E_EDGES, 128), jnp.float32),
                  jax.ShapeDtypeStruct((E_EDGES, 128), jnp.float32)],
        mesh=mesh,
        scratch_types=[
            pltpu.VMEM((_EW,), jnp.int32),
            pltpu.VMEM((_EW,), jnp.int32),
            pltpu.VMEM((_GC, 128), jnp.float32),
            pltpu.VMEM((_GC, 128), jnp.float32),
            pltpu.SemaphoreType.DMA,
            pltpu.SemaphoreType.DMA,
            pltpu.SemaphoreType.DMA,
            pltpu.SemaphoreType.DMA,
        ],
    )
    def k(xl_hbm, xr_hbm, src_hbm, dst_hbm, xs_hbm, xd_hbm,
          sidx, didx, srows, drows, sem1, sem2, semw1, semw2):
        wid = lax.axis_index("s") * _NC + lax.axis_index("c")
        w0 = wid * _EW
        pltpu.sync_copy(src_hbm.at[pl.ds(w0, _EW)], sidx)
        pltpu.sync_copy(dst_hbm.at[pl.ds(w0, _EW)], didx)

        def body(i, _):
            base = w0 + i * _GC

            @pl.when(i > 0)
            def _():
                pltpu.make_async_copy(srows, xs_hbm.at[pl.ds(w0, _GC)], semw1).wait()
                pltpu.make_async_copy(drows, xd_hbm.at[pl.ds(w0, _GC)], semw2).wait()

            a = pltpu.async_copy(xl_hbm.at[sidx.at[pl.ds(i * _GC, _GC)]], srows, sem1)
            b = pltpu.async_copy(xr_hbm.at[didx.at[pl.ds(i * _GC, _GC)]], drows, sem2)
            a.wait()
            pltpu.async_copy(srows, xs_hbm.at[pl.ds(base, _GC)], semw1)
            b.wait()
            pltpu.async_copy(drows, xd_hbm.at[pl.ds(base, _GC)], semw2)
            return 0

        lax.fori_loop(0, _EW // _GC, body, 0)
        pltpu.make_async_copy(srows, xs_hbm.at[pl.ds(w0, _GC)], semw1).wait()
        pltpu.make_async_copy(drows, xd_hbm.at[pl.ds(w0, _GC)], semw2).wait()

    return k(xl, xr, src, dst)


def _sc_scatter(R0, R1, dst, zrows):
    mesh = plsc.VectorSubcoreMesh(core_axis_name="c", subcore_axis_name="s")

    @functools.partial(
        pl.kernel,
        out_type=[jax.ShapeDtypeStruct((N_NODES, ACC_W), jnp.float32),
                  jax.ShapeDtypeStruct((N_NODES, ACC_W), jnp.float32)],
        mesh=mesh,
        scratch_types=[
            pltpu.VMEM((_SCC,), jnp.int32),
            pltpu.VMEM((_SCC, ACC_W), jnp.float32),
            pltpu.VMEM_SHARED((N_NODES, ACC_W), jnp.float32),
            pltpu.SemaphoreType.DMA,
        ],
        compiler_params=pltpu.CompilerParams(use_tc_tiling_on_sc=False),
    )
    def k(r0_hbm, r1_hbm, dst_hbm, z_hbm, a0_hbm, a1_hbm, idxb, valb, acc, sem):
        sc = lax.axis_index("c")
        t = lax.axis_index("s")

        @pl.when(t < 15)
        def _():
            pltpu.sync_copy(z_hbm, acc.at[pl.ds(t * _STR, _STR)])

        @pl.when(t == 15)
        def _():
            pltpu.sync_copy(z_hbm.at[pl.ds(0, _STR_LAST)],
                            acc.at[pl.ds(15 * _STR, _STR_LAST)])

        plsc.subcore_barrier()

        def body(i, _):
            base = t * _ET + i * _SCC
            pltpu.sync_copy(dst_hbm.at[pl.ds(base, _SCC)], idxb)

            @pl.when(sc == 0)
            def _():
                pltpu.sync_copy(r0_hbm.at[pl.ds(base, _SCC)], valb)

            @pl.when(sc == 1)
            def _():
                pltpu.sync_copy(r1_hbm.at[pl.ds(base, _SCC)], valb)

            pltpu.sync_copy(valb, acc.at[idxb], add=True)
            return 0

        lax.fori_loop(0, _ET // _SCC, body, 0)
        plsc.subcore_barrier()

        def wb(out_hbm):
            @pl.when(t < 15)
            def _():
                pltpu.sync_copy(acc.at[pl.ds(t * _STR, _STR)],
                                out_hbm.at[pl.ds(t * _STR, _STR)])

            @pl.when(t == 15)
            def _():
                pltpu.sync_copy(acc.at[pl.ds(15 * _STR, _STR_LAST)],
                                out_hbm.at[pl.ds(15 * _STR, _STR_LAST)])

        @pl.when(sc == 0)
        def _():
            wb(a0_hbm)

        @pl.when(sc == 1)
        def _():
            wb(a1_hbm)

    return k(R0, R1, dst, zrows)


def _gat_layer(x, src, dst, ea, zrows, Wl, Wr, We, att, bias, lW, lb):
    xl, xr = _mm2(x, Wl, Wr)
    XS, XD = _sc_gather2(xl, xr, src, dst)
    R0, R1 = _edge(XS, XD, ea, We, att)
    acc0, acc1 = _sc_scatter(R0, R1, dst, zrows)
    return _finish(acc0, acc1, bias, lW, lb)


def _two_blocks(x_m, mg_ei, mg_ea, mg_layers, x_p, pg_ei, pg_ea, pg_layers, zrows):
    m_src, m_dst = mg_ei[0], mg_ei[1]
    p_src, p_dst = pg_ei[0], pg_ei[1]
    for lm, lp in zip(mg_layers, pg_layers):
        (mWl, mWr, mWe, matt, mb, mlW, mlb) = lm
        (pWl, pWr, pWe, patt, pb, plW, plb) = lp
        m_xl, m_xr = _mm2(x_m, mWl, mWr)
        p_xl, p_xr = _mm2(x_p, pWl, pWr)
        m_XS, m_XD = _sc_gather2(m_xl, m_xr, m_src, m_dst)
        p_XS, p_XD = _sc_gather2(p_xl, p_xr, p_src, p_dst)
        m_R0, m_R1 = _edge(m_XS, m_XD, mg_ea, mWe, matt)
        p_R0, p_R1 = _edge(p_XS, p_XD, pg_ea, pWe, patt)
        m_a0, m_a1 = _sc_scatter(m_R0, m_R1, m_dst, zrows)
        p_a0, p_a1 = _sc_scatter(p_R0, p_R1, p_dst, zrows)
        x_m = _finish(m_a0, m_a1, mb, mlW, mlb)
        x_p = _finish(p_a0, p_a1, pb, plW, plb)
    return x_m, x_p


def kernel(mg_x, mg_edge_index, mg_edge_attr, pg_x, pg_edge_index, pg_edge_attr,
           mg_Wl0, mg_Wr0, mg_We0, mg_att0, mg_b0, mg_lW0, mg_lb0,
           mg_Wl1, mg_Wr1, mg_We1, mg_att1, mg_b1, mg_lW1, mg_lb1,
           pg_Wl0, pg_Wr0, pg_We0, pg_att0, pg_b0, pg_lW0, pg_lb0,
           pg_Wl1, pg_Wr1, pg_We1, pg_att1, pg_b1, pg_lW1, pg_lb1,
           f1W, f1b, f2W, f2b):
    mg_layers = [
        (mg_Wl0, mg_Wr0, mg_We0, mg_att0, mg_b0, mg_lW0, mg_lb0),
        (mg_Wl1, mg_Wr1, mg_We1, mg_att1, mg_b1, mg_lW1, mg_lb1),
    ]
    pg_layers = [
        (pg_Wl0, pg_Wr0, pg_We0, pg_att0, pg_b0, pg_lW0, pg_lb0),
        (pg_Wl1, pg_Wr1, pg_We1, pg_att1, pg_b1, pg_lW1, pg_lb1),
    ]
    zrows = jnp.zeros((_STR, ACC_W), jnp.float32)
    x_m, x_p = _two_blocks(mg_x, mg_edge_index, mg_edge_attr, mg_layers,
                           pg_x, pg_edge_index, pg_edge_attr, pg_layers, zrows)
    return _head(x_m, x_p, f1W, f1b, f2W, f2b)


# speedup vs baseline: 1.1299x; 1.0081x over previous
import functools
import numpy as np
import jax
import jax.numpy as jnp
from jax import lax
from jax.experimental import pallas as pl
from jax.experimental.pallas import tpu as pltpu
from jax.experimental.pallas import tpu_sc as plsc

N_NODES = 10000
E_EDGES = 160000
OUT, HEADS, CPH = 256, 8, 32
D_EDGE = 16
ACC_W = 144

_NC, _NS = 2, 16
_NW = _NC * _NS
_GC = 200
_EW = E_EDGES // _NW
_SCC = 200
_ET = E_EDGES // _NS
_STR = 640
_STR_LAST = N_NODES - 15 * _STR

_BLK = np.zeros((OUT, HEADS), np.float32)
for _h in range(HEADS):
    _BLK[_h * CPH:(_h + 1) * CPH, _h] = 1.0
_BLKT = _BLK.T.copy()
_P0 = np.zeros((HEADS, 16), np.float32)
_P1 = np.zeros((HEADS, 16), np.float32)
for _h in range(4):
    _P0[_h, _h] = 1.0
    _P1[_h + 4, _h] = 1.0
_B16 = np.zeros((16, 128), np.float32)
for _h in range(4):
    _B16[_h, _h * CPH:(_h + 1) * CPH] = 1.0


def _pack_bf16_pair(y):
    lo = lax.bitcast_convert_type(y[:, :128].astype(jnp.bfloat16), jnp.uint16)
    hi = lax.bitcast_convert_type(y[:, 128:].astype(jnp.bfloat16), jnp.uint16)
    word = lo.astype(jnp.uint32) | (hi.astype(jnp.uint32) << 16)
    return lax.bitcast_convert_type(word, jnp.float32)


def _unpack_bf16_pair(p):
    u = lax.bitcast_convert_type(p, jnp.uint32)
    lo = lax.bitcast_convert_type((u & 0xFFFF).astype(jnp.uint16), jnp.bfloat16)
    hi = lax.bitcast_convert_type((u >> 16).astype(jnp.uint16), jnp.bfloat16)
    return lo.astype(jnp.float32), hi.astype(jnp.float32)


def _mm2_body(x_ref, wl_ref, wr_ref, ol_ref, or_ref):
    x = x_ref[...]
    ol_ref[...] = _pack_bf16_pair(jnp.dot(x, wl_ref[...], preferred_element_type=jnp.float32))
    or_ref[...] = _pack_bf16_pair(jnp.dot(x, wr_ref[...], preferred_element_type=jnp.float32))


def _mm2(x, Wl, Wr):
    M, K = x.shape
    BM = 1000
    return pl.pallas_call(
        _mm2_body,
        grid=(M // BM,),
        in_specs=[
            pl.BlockSpec((BM, K), lambda i: (i, 0)),
            pl.BlockSpec((K, OUT), lambda i: (0, 0)),
            pl.BlockSpec((K, OUT), lambda i: (0, 0)),
        ],
        out_specs=[
            pl.BlockSpec((BM, 128), lambda i: (i, 0)),
            pl.BlockSpec((BM, 128), lambda i: (i, 0)),
        ],
        out_shape=[
            jax.ShapeDtypeStruct((M, 128), jnp.float32),
            jax.ShapeDtypeStruct((M, 128), jnp.float32),
        ],
    )(x, Wl, Wr)


def _edge_body(xs_ref, xd_ref, ea_ref, we_ref, a_ref, bt_ref, p0_ref, p1_ref,
               r0_ref, r1_ref):
    xs0, xs1 = _unpack_bf16_pair(xs_ref[...])
    xd0, xd1 = _unpack_bf16_pair(xd_ref[...])
    e = jnp.dot(ea_ref[...], we_ref[...], preferred_element_type=jnp.float32)
    m0 = xs0 + xd0 + e[:, :128]
    m1 = xs1 + xd1 + e[:, 128:]
    m0 = jnp.where(m0 >= 0, m0, 0.2 * m0)
    m1 = jnp.where(m1 >= 0, m1, 0.2 * m1)
    alpha = (jnp.dot(m0, a_ref[:128, :], preferred_element_type=jnp.float32)
             + jnp.dot(m1, a_ref[128:, :], preferred_element_type=jnp.float32))
    w = jnp.exp(alpha)
    wb0 = jnp.dot(w, bt_ref[:, :128], preferred_element_type=jnp.float32)
    wb1 = jnp.dot(w, bt_ref[:, 128:], preferred_element_type=jnp.float32)
    w0 = jnp.dot(w, p0_ref[...], preferred_element_type=jnp.float32)
    w1 = jnp.dot(w, p1_ref[...], preferred_element_type=jnp.float32)
    r0_ref[...] = jnp.concatenate([xs0 * wb0, w0], axis=1)
    r1_ref[...] = jnp.concatenate([xs1 * wb1, w1], axis=1)


def _edge(XS, XD, ea, We, att):
    BE = 4000
    A = att.reshape(-1)[:, None] * jnp.asarray(_BLK)
    return pl.pallas_call(
        _edge_body,
        grid=(E_EDGES // BE,),
        in_specs=[
            pl.BlockSpec((BE, 128), lambda i: (i, 0)),
            pl.BlockSpec((BE, 128), lambda i: (i, 0)),
            pl.BlockSpec((BE, D_EDGE), lambda i: (i, 0)),
            pl.BlockSpec((D_EDGE, OUT), lambda i: (0, 0)),
            pl.BlockSpec((OUT, HEADS), lambda i: (0, 0)),
            pl.BlockSpec((HEADS, OUT), lambda i: (0, 0)),
            pl.BlockSpec((HEADS, 16), lambda i: (0, 0)),
            pl.BlockSpec((HEADS, 16), lambda i: (0, 0)),
        ],
        out_specs=[
            pl.BlockSpec((BE, ACC_W), lambda i: (i, 0)),
            pl.BlockSpec((BE, ACC_W), lambda i: (i, 0)),
        ],
        out_shape=[
            jax.ShapeDtypeStruct((E_EDGES, ACC_W), jnp.float32),
            jax.ShapeDtypeStruct((E_EDGES, ACC_W), jnp.float32),
        ],
    )(XS, XD, ea, We, A, jnp.asarray(_BLKT), jnp.asarray(_P0), jnp.asarray(_P1))


def _finish_body(a0_ref, a1_ref, b16_ref, b_ref, lw_ref, lb_ref, o_ref):
    db0 = jnp.dot(a0_ref[:, 128:144], b16_ref[...], preferred_element_type=jnp.float32)
    db1 = jnp.dot(a1_ref[:, 128:144], b16_ref[...], preferred_element_type=jnp.float32)
    x0 = a0_ref[:, 0:128] / (db0 + 1e-16)
    x1 = a1_ref[:, 0:128] / (db1 + 1e-16)
    x = jnp.concatenate([x0, x1], axis=1) + b_ref[...]
    y = jnp.dot(x, lw_ref[...], preferred_element_type=jnp.float32) + lb_ref[...]
    o_ref[...] = x + jnp.where(y > 0, y, jnp.exp(jnp.minimum(y, 0.0)) - 1.0)


def _finish(acc0, acc1, bias, lW, lb):
    BM = 1000
    return pl.pallas_call(
        _finish_body,
        grid=(N_NODES // BM,),
        in_specs=[
            pl.BlockSpec((BM, ACC_W), lambda i: (i, 0)),
            pl.BlockSpec((BM, ACC_W), lambda i: (i, 0)),
            pl.BlockSpec((16, 128), lambda i: (0, 0)),
            pl.BlockSpec((1, OUT), lambda i: (0, 0)),
            pl.BlockSpec((OUT, OUT), lambda i: (0, 0)),
            pl.BlockSpec((1, OUT), lambda i: (0, 0)),
        ],
        out_specs=pl.BlockSpec((BM, OUT), lambda i: (i, 0)),
        out_shape=jax.ShapeDtypeStruct((N_NODES, OUT), jnp.float32),
    )(acc0, acc1, jnp.asarray(_B16), bias.reshape(1, OUT), lW, lb.reshape(1, OUT))


def _head_body(xm_ref, xp_ref, f1m_ref, f1p_ref, f1b_ref, f2w_ref, f2b_ref,
               o_ref, sm_ref, sp_ref):
    i = pl.program_id(0)

    @pl.when(i == 0)
    def _():
        sm_ref[...] = jnp.zeros_like(sm_ref)
        sp_ref[...] = jnp.zeros_like(sp_ref)

    sm_ref[...] += jnp.sum(xm_ref[...], 0, keepdims=True)
    sp_ref[...] += jnp.sum(xp_ref[...], 0, keepdims=True)

    @pl.when(i == pl.num_programs(0) - 1)
    def _():
        z = (jnp.dot(sm_ref[...], f1m_ref[...], preferred_element_type=jnp.float32)
             + jnp.dot(sp_ref[...], f1p_ref[...], preferred_element_type=jnp.float32)
             + f1b_ref[...])
        z = jnp.where(z > 0, z, jnp.exp(jnp.minimum(z, 0.0)) - 1.0)
        t = jnp.dot(z, f2w_ref[...], preferred_element_type=jnp.float32) + f2b_ref[...]
        o_ref[...] = 1.0 / (1.0 + jnp.exp(-t[0:1, 0:1]))


def _head(x_m, x_p, f1W, f1b, f2W, f2b):
    BM = 400
    return pl.pallas_call(
        _head_body,
        grid=(N_NODES // BM,),
        in_specs=[
            pl.BlockSpec((BM, OUT), lambda i: (i, 0)),
            pl.BlockSpec((BM, OUT), lambda i: (i, 0)),
            pl.BlockSpec((OUT, OUT), lambda i: (0, 0)),
            pl.BlockSpec((OUT, OUT), lambda i: (0, 0)),
            pl.BlockSpec((1, OUT), lambda i: (0, 0)),
            pl.BlockSpec((OUT, 1), lambda i: (0, 0)),
            pl.BlockSpec((1, 1), lambda i: (0, 0)),
        ],
        out_specs=pl.BlockSpec((1, 1), lambda i: (0, 0)),
        out_shape=jax.ShapeDtypeStruct((1, 1), jnp.float32),
        scratch_shapes=[
            pltpu.VMEM((1, OUT), jnp.float32),
            pltpu.VMEM((1, OUT), jnp.float32),
        ],
    )(x_m, x_p, f1W[:OUT], f1W[OUT:], f1b.reshape(1, OUT), f2W, f2b.reshape(1, 1))


def _sc_gather2(xl, xr, src, dst):
    mesh = plsc.VectorSubcoreMesh(core_axis_name="c", subcore_axis_name="s")

    @functools.partial(
        pl.kernel,
        out_type=[jax.ShapeDtypeStruct((E_EDGES, 128), jnp.float32),
                  jax.ShapeDtypeStruct((E_EDGES, 128), jnp.float32)],
        mesh=mesh,
        scratch_types=[
            pltpu.VMEM((_EW,), jnp.int32),
            pltpu.VMEM((_EW,), jnp.int32),
            pltpu.VMEM((_GC, 128), jnp.float32),
            pltpu.VMEM((_GC, 128), jnp.float32),
            pltpu.SemaphoreType.DMA,
            pltpu.SemaphoreType.DMA,
            pltpu.SemaphoreType.DMA,
            pltpu.SemaphoreType.DMA,
        ],
    )
    def k(xl_hbm, xr_hbm, src_hbm, dst_hbm, xs_hbm, xd_hbm,
          sidx, didx, srows, drows, sem1, sem2, semw1, semw2):
        wid = lax.axis_index("s") * _NC + lax.axis_index("c")
        w0 = wid * _EW
        pltpu.sync_copy(src_hbm.at[pl.ds(w0, _EW)], sidx)
        pltpu.sync_copy(dst_hbm.at[pl.ds(w0, _EW)], didx)

        def body(i, _):
            base = w0 + i * _GC

            @pl.when(i > 0)
            def _():
                pltpu.make_async_copy(srows, xs_hbm.at[pl.ds(w0, _GC)], semw1).wait()
                pltpu.make_async_copy(drows, xd_hbm.at[pl.ds(w0, _GC)], semw2).wait()

            a = pltpu.async_copy(xl_hbm.at[sidx.at[pl.ds(i * _GC, _GC)]], srows, sem1)
            b = pltpu.async_copy(xr_hbm.at[didx.at[pl.ds(i * _GC, _GC)]], drows, sem2)
            a.wait()
            pltpu.async_copy(srows, xs_hbm.at[pl.ds(base, _GC)], semw1)
            b.wait()
            pltpu.async_copy(drows, xd_hbm.at[pl.ds(base, _GC)], semw2)
            return 0

        lax.fori_loop(0, _EW // _GC, body, 0)
        pltpu.make_async_copy(srows, xs_hbm.at[pl.ds(w0, _GC)], semw1).wait()
        pltpu.make_async_copy(drows, xd_hbm.at[pl.ds(w0, _GC)], semw2).wait()

    return k(xl, xr, src, dst)


def _sc_scatter(R0, R1, dst, zrows):
    mesh = plsc.VectorSubcoreMesh(core_axis_name="c", subcore_axis_name="s")

    @functools.partial(
        pl.kernel,
        out_type=[jax.ShapeDtypeStruct((N_NODES, ACC_W), jnp.float32),
                  jax.ShapeDtypeStruct((N_NODES, ACC_W), jnp.float32)],
        mesh=mesh,
        scratch_types=[
            pltpu.VMEM((_ET // _SCC, _SCC), jnp.int32),
            pltpu.VMEM((_SCC, ACC_W), jnp.float32),
            pltpu.VMEM_SHARED((N_NODES, ACC_W), jnp.float32),
            pltpu.SemaphoreType.DMA,
        ],
        compiler_params=pltpu.CompilerParams(use_tc_tiling_on_sc=False),
    )
    def k(r0_hbm, r1_hbm, dst_hbm, z_hbm, a0_hbm, a1_hbm, idxb, valb, acc, sem):
        sc = lax.axis_index("c")
        t = lax.axis_index("s")
        nck = _ET // _SCC
        pltpu.sync_copy(dst_hbm.at[pl.ds(t * nck, nck)], idxb)

        @pl.when(t < 15)
        def _():
            pltpu.sync_copy(z_hbm, acc.at[pl.ds(t * _STR, _STR)])

        @pl.when(t == 15)
        def _():
            pltpu.sync_copy(z_hbm.at[pl.ds(0, _STR_LAST)],
                            acc.at[pl.ds(15 * _STR, _STR_LAST)])

        plsc.subcore_barrier()

        def body(i, _):
            base = t * _ET + i * _SCC

            @pl.when(sc == 0)
            def _():
                pltpu.sync_copy(r0_hbm.at[pl.ds(base, _SCC)], valb)

            @pl.when(sc == 1)
            def _():
                pltpu.sync_copy(r1_hbm.at[pl.ds(base, _SCC)], valb)

            pltpu.sync_copy(valb, acc.at[idxb.at[i]], add=True)
            return 0

        lax.fori_loop(0, _ET // _SCC, body, 0)
        plsc.subcore_barrier()

        def wb(out_hbm):
            @pl.when(t < 15)
            def _():
                pltpu.sync_copy(acc.at[pl.ds(t * _STR, _STR)],
                                out_hbm.at[pl.ds(t * _STR, _STR)])

            @pl.when(t == 15)
            def _():
                pltpu.sync_copy(acc.at[pl.ds(15 * _STR, _STR_LAST)],
                                out_hbm.at[pl.ds(15 * _STR, _STR_LAST)])

        @pl.when(sc == 0)
        def _():
            wb(a0_hbm)

        @pl.when(sc == 1)
        def _():
            wb(a1_hbm)

    return k(R0, R1, dst.reshape(E_EDGES // _SCC, _SCC), zrows)


def _gat_layer(x, src, dst, ea, zrows, Wl, Wr, We, att, bias, lW, lb):
    xl, xr = _mm2(x, Wl, Wr)
    XS, XD = _sc_gather2(xl, xr, src, dst)
    R0, R1 = _edge(XS, XD, ea, We, att)
    acc0, acc1 = _sc_scatter(R0, R1, dst, zrows)
    return _finish(acc0, acc1, bias, lW, lb)


def _two_blocks(x_m, mg_ei, mg_ea, mg_layers, x_p, pg_ei, pg_ea, pg_layers, zrows):
    m_src, m_dst = mg_ei[0], mg_ei[1]
    p_src, p_dst = pg_ei[0], pg_ei[1]
    for lm, lp in zip(mg_layers, pg_layers):
        (mWl, mWr, mWe, matt, mb, mlW, mlb) = lm
        (pWl, pWr, pWe, patt, pb, plW, plb) = lp
        m_xl, m_xr = _mm2(x_m, mWl, mWr)
        p_xl, p_xr = _mm2(x_p, pWl, pWr)
        m_XS, m_XD = _sc_gather2(m_xl, m_xr, m_src, m_dst)
        p_XS, p_XD = _sc_gather2(p_xl, p_xr, p_src, p_dst)
        m_R0, m_R1 = _edge(m_XS, m_XD, mg_ea, mWe, matt)
        p_R0, p_R1 = _edge(p_XS, p_XD, pg_ea, pWe, patt)
        m_a0, m_a1 = _sc_scatter(m_R0, m_R1, m_dst, zrows)
        p_a0, p_a1 = _sc_scatter(p_R0, p_R1, p_dst, zrows)
        x_m = _finish(m_a0, m_a1, mb, mlW, mlb)
        x_p = _finish(p_a0, p_a1, pb, plW, plb)
    return x_m, x_p


def kernel(mg_x, mg_edge_index, mg_edge_attr, pg_x, pg_edge_index, pg_edge_attr,
           mg_Wl0, mg_Wr0, mg_We0, mg_att0, mg_b0, mg_lW0, mg_lb0,
           mg_Wl1, mg_Wr1, mg_We1, mg_att1, mg_b1, mg_lW1, mg_lb1,
           pg_Wl0, pg_Wr0, pg_We0, pg_att0, pg_b0, pg_lW0, pg_lb0,
           pg_Wl1, pg_Wr1, pg_We1, pg_att1, pg_b1, pg_lW1, pg_lb1,
           f1W, f1b, f2W, f2b):
    mg_layers = [
        (mg_Wl0, mg_Wr0, mg_We0, mg_att0, mg_b0, mg_lW0, mg_lb0),
        (mg_Wl1, mg_Wr1, mg_We1, mg_att1, mg_b1, mg_lW1, mg_lb1),
    ]
    pg_layers = [
        (pg_Wl0, pg_Wr0, pg_We0, pg_att0, pg_b0, pg_lW0, pg_lb0),
        (pg_Wl1, pg_Wr1, pg_We1, pg_att1, pg_b1, pg_lW1, pg_lb1),
    ]
    zrows = jnp.zeros((_STR, ACC_W), jnp.float32)
    x_m, x_p = _two_blocks(mg_x, mg_edge_index, mg_edge_attr, mg_layers,
                           pg_x, pg_edge_index, pg_edge_attr, pg_layers, zrows)
    return _head(x_m, x_p, f1W, f1b, f2W, f2b)
